# baseline (speedup 1.0000x reference)
import functools
import math

import jax
import jax.numpy as jnp
from jax import lax
from jax.experimental import pallas as pl
from jax.experimental.pallas import tpu as pltpu
from jax.experimental.pallas import tpu_sc as plsc

NC = 2
NS = 16
LANES = 16
CH = 128
MACRO = 2
HALF = 32

_f32 = jnp.float32


def _ceil_to(x, m):
  return (x + m - 1) // m * m


def _zero_fill(buf_ref, nrows, ncols):
  per_row = ncols // LANES

  def body(i, _):
    r = i // per_row
    c = (i % per_row) * LANES
    buf_ref[r, pl.ds(c, LANES)] = jnp.zeros((LANES,), _f32)
    return 0

  lax.fori_loop(0, nrows * per_row, body, 0)


def _const_fill(buf_ref, nrows, ncols, val):
  per_row = ncols // LANES

  def body(i, _):
    r = i // per_row
    c = (i % per_row) * LANES
    buf_ref[r, pl.ds(c, LANES)] = jnp.full((LANES,), val, _f32)
    return 0

  lax.fori_loop(0, nrows * per_row, body, 0)


def _sc_hist(np_rows, n_chunks):
  mesh = plsc.VectorSubcoreMesh(core_axis_name="c", subcore_axis_name="s",
                                num_cores=NC, num_subcores=NS)
  acc_rows = np_rows + CH
  out_chunks = np_rows // CH

  @functools.partial(
      pl.kernel,
      mesh=mesh,
      compiler_params=pltpu.CompilerParams(use_tc_tiling_on_sc=False),
      out_type=jax.ShapeDtypeStruct((NC, np_rows), _f32),
      scratch_types=[
          pltpu.VMEM_SHARED((acc_rows,), _f32),
          pltpu.VMEM((CH,), jnp.int32),
          pltpu.VMEM((CH,), _f32),
          pltpu.VMEM((CH,), _f32),
      ],
  )
  def k(dst_hbm, out_hbm, acc, idx_v, ones_v, buf_v):
    c = lax.axis_index("c")
    s = lax.axis_index("s")

    def fill(i, _):
      buf_v[pl.ds(i * LANES, LANES)] = jnp.zeros((LANES,), _f32)
      ones_v[pl.ds(i * LANES, LANES)] = jnp.ones((LANES,), _f32)
      return 0

    lax.fori_loop(0, CH // LANES, fill, 0)

    nz = acc_rows // CH

    def zbody(i, _):
      pltpu.sync_copy(buf_v, acc.at[pl.ds((i * NS + s) * CH, CH)])
      return 0

    lax.fori_loop(0, nz // NS, zbody, 0)

    @pl.when(s < nz % NS)
    def _():
      pltpu.sync_copy(buf_v, acc.at[pl.ds(((nz // NS) * NS + s) * CH, CH)])

    plsc.subcore_barrier()

    half = n_chunks // NC

    def ebody(i, _):
      ch = c * half + i * NS + s
      pltpu.sync_copy(dst_hbm.at[ch], idx_v)
      pltpu.sync_copy(ones_v, acc.at[idx_v], add=True)
      return 0

    lax.fori_loop(0, half // NS, ebody, 0)

    @pl.when(s < half % NS)
    def _():
      ch = c * half + (half // NS) * NS + s
      pltpu.sync_copy(dst_hbm.at[ch], idx_v)
      pltpu.sync_copy(ones_v, acc.at[idx_v], add=True)

    plsc.subcore_barrier()

    def obody(i, _):
      r = (i * NS + s) * CH
      pltpu.sync_copy(acc.at[pl.ds(r, CH)], buf_v)
      pltpu.sync_copy(buf_v, out_hbm.at[c, pl.ds(r, CH)])
      return 0

    lax.fori_loop(0, out_chunks // NS, obody, 0)

    @pl.when(s < out_chunks % NS)
    def _():
      r = ((out_chunks // NS) * NS + s) * CH
      pltpu.sync_copy(acc.at[pl.ds(r, CH)], buf_v)
      pltpu.sync_copy(buf_v, out_hbm.at[c, pl.ds(r, CH)])

  return k


def _sc_agg(np_rows, n_chunks):
  mesh = plsc.VectorSubcoreMesh(core_axis_name="c", subcore_axis_name="s",
                                num_cores=NC, num_subcores=NS)
  acc_rows = np_rows + CH
  out_chunks = np_rows // CH
  n_pair = n_chunks // (2 * MACRO)

  @functools.partial(
      pl.kernel,
      mesh=mesh,
      compiler_params=pltpu.CompilerParams(use_tc_tiling_on_sc=False),
      out_type=jax.ShapeDtypeStruct((NC, np_rows, HALF), _f32),
      scratch_types=[
          pltpu.VMEM_SHARED((acc_rows, HALF), _f32),
          pltpu.VMEM((MACRO, CH), jnp.int32),
          pltpu.VMEM((MACRO, CH), jnp.int32),
          pltpu.VMEM((MACRO, CH), jnp.int32),
          pltpu.VMEM((MACRO, CH), jnp.int32),
          pltpu.VMEM((MACRO * CH, HALF), _f32),
          pltpu.VMEM((MACRO * CH, HALF), _f32),
          pltpu.SemaphoreType.DMA,
          pltpu.SemaphoreType.DMA,
          pltpu.SemaphoreType.DMA,
          pltpu.SemaphoreType.DMA,
      ],
  )
  def k(h_hbm, src_hbm, dst_hbm, out_hbm, acc, src_a, dst_a, src_b, dst_b,
        rows_a, rows_b, gsem_a, ssem_a, gsem_b, ssem_b):
    c = lax.axis_index("c")
    s = lax.axis_index("s")
    buf_v = rows_a.at[pl.ds(0, CH)]
    _zero_fill(rows_a, CH, HALF)

    nz = acc_rows // CH

    def zbody(i, _):
      pltpu.sync_copy(buf_v, acc.at[pl.ds((i * NS + s) * CH, CH)])
      return 0

    lax.fori_loop(0, nz // NS, zbody, 0)

    @pl.when(s < nz % NS)
    def _():
      pltpu.sync_copy(buf_v, acc.at[pl.ds(((nz // NS) * NS + s) * CH, CH)])

    plsc.subcore_barrier()

    def ebody(i, _):
      m = (i * NS + s) * 2 * MACRO
      pltpu.sync_copy(src_hbm.at[pl.ds(m, MACRO)], src_a)
      pltpu.sync_copy(dst_hbm.at[pl.ds(m, MACRO)], dst_a)
      ga = [pltpu.async_copy(h_hbm.at[c].at[src_a.at[j]],
                             rows_a.at[pl.ds(j * CH, CH)], gsem_a)
            for j in range(MACRO)]
      pltpu.sync_copy(src_hbm.at[pl.ds(m + MACRO, MACRO)], src_b)
      pltpu.sync_copy(dst_hbm.at[pl.ds(m + MACRO, MACRO)], dst_b)
      gb = [pltpu.async_copy(h_hbm.at[c].at[src_b.at[j]],
                             rows_b.at[pl.ds(j * CH, CH)], gsem_b)
            for j in range(MACRO)]
      for d in ga:
        d.wait()
      sa = [pltpu.async_copy(rows_a.at[pl.ds(j * CH, CH)],
                             acc.at[dst_a.at[j]], ssem_a, add=True)
            for j in range(MACRO)]
      for d in gb:
        d.wait()
      sb = [pltpu.async_copy(rows_b.at[pl.ds(j * CH, CH)],
                             acc.at[dst_b.at[j]], ssem_b, add=True)
            for j in range(MACRO)]
      for d in sa:
        d.wait()
      for d in sb:
        d.wait()
      return 0

    lax.fori_loop(0, n_pair // NS, ebody, 0)

    plsc.subcore_barrier()

    def obody(i, _):
      r = (i * NS + s) * CH
      pltpu.sync_copy(acc.at[pl.ds(r, CH)], buf_v)
      pltpu.sync_copy(buf_v, out_hbm.at[c, pl.ds(r, CH)])
      return 0

    lax.fori_loop(0, out_chunks // NS, obody, 0)

    @pl.when(s < out_chunks % NS)
    def _():
      r = ((out_chunks // NS) * NS + s) * CH
      pltpu.sync_copy(acc.at[pl.ds(r, CH)], buf_v)
      pltpu.sync_copy(buf_v, out_hbm.at[c, pl.ds(r, CH)])

  return k


def _sc_pool(np_rows, pool_rows):
  mesh = plsc.VectorSubcoreMesh(core_axis_name="c", subcore_axis_name="s",
                                num_cores=NC, num_subcores=NS)
  n_chunks = np_rows // CH
  out_chunks = pool_rows // CH

  @functools.partial(
      pl.kernel,
      mesh=mesh,
      compiler_params=pltpu.CompilerParams(use_tc_tiling_on_sc=False),
      out_type=jax.ShapeDtypeStruct((NC, pool_rows, HALF), _f32),
      scratch_types=[
          pltpu.VMEM_SHARED((pool_rows, HALF), _f32),
          pltpu.VMEM((CH,), jnp.int32),
          pltpu.VMEM((CH, HALF), _f32),
          pltpu.VMEM((CH, HALF), _f32),
      ],
  )
  def k(h_hbm, b_hbm, out_hbm, acc, idx_v, vals_v, buf_v):
    c = lax.axis_index("c")
    s = lax.axis_index("s")
    _zero_fill(buf_v, CH, HALF)

    @pl.when(s < out_chunks)
    def _():
      pltpu.sync_copy(buf_v, acc.at[pl.ds(s * CH, CH)])

    plsc.subcore_barrier()

    def ebody(i, _):
      ch = i * NS + s
      pltpu.sync_copy(b_hbm.at[ch], idx_v)
      pltpu.sync_copy(h_hbm.at[c, pl.ds(ch * CH, CH)], vals_v)
      pltpu.sync_copy(vals_v, acc.at[idx_v], add=True)
      return 0

    lax.fori_loop(0, n_chunks // NS, ebody, 0)

    @pl.when(s < n_chunks % NS)
    def _():
      ch = (n_chunks // NS) * NS + s
      pltpu.sync_copy(b_hbm.at[ch], idx_v)
      pltpu.sync_copy(h_hbm.at[c, pl.ds(ch * CH, CH)], vals_v)
      pltpu.sync_copy(vals_v, acc.at[idx_v], add=True)

    plsc.subcore_barrier()

    @pl.when(s < out_chunks)
    def _():
      pltpu.sync_copy(acc.at[pl.ds(s * CH, CH)], buf_v)
      pltpu.sync_copy(buf_v, out_hbm.at[c, pl.ds(s * CH, CH)])

  return k


def _sc_gather_rows(np_rows, d):
  mesh = plsc.VectorSubcoreMesh(core_axis_name="c", subcore_axis_name="s",
                                num_cores=NC, num_subcores=NS)
  n_chunks = np_rows // CH

  @functools.partial(
      pl.kernel,
      mesh=mesh,
      compiler_params=pltpu.CompilerParams(use_tc_tiling_on_sc=False),
      out_type=jax.ShapeDtypeStruct((np_rows, d), _f32),
      scratch_types=[
          pltpu.VMEM((CH,), jnp.int32),
          pltpu.VMEM((CH, d), _f32),
          pltpu.SemaphoreType.DMA,
      ],
  )
  def k(tab_hbm, idx_hbm, out_hbm, idx_v, rows_v, sem):
    c = lax.axis_index("c")
    s = lax.axis_index("s")
    w = s * NC + c
    nw = NC * NS

    def body(i, _):
      ch = i * nw + w
      pltpu.sync_copy(idx_hbm.at[ch], idx_v)
      pltpu.async_copy(tab_hbm.at[idx_v], rows_v, sem).wait()
      pltpu.sync_copy(rows_v, out_hbm.at[pl.ds(ch * CH, CH)])
      return 0

    lax.fori_loop(0, n_chunks // nw, body, 0)

  return k


_BLK = 256


def _tc_invdeg(np_rows):
  blk = 1024

  def body(d0_ref, d1_ref, o_ref):
    d = d0_ref[...] + d1_ref[...]
    o_ref[...] = 1.0 / jnp.maximum(d, 1.0)

  return pl.pallas_call(
      body,
      grid=(np_rows // blk,),
      in_specs=[pl.BlockSpec((blk,), lambda i: (i,)),
                pl.BlockSpec((blk,), lambda i: (i,))],
      out_specs=pl.BlockSpec((blk,), lambda i: (i,)),
      out_shape=jax.ShapeDtypeStruct((np_rows,), _f32),
  )


def _tc_frag_in(np_rows, in_dim):
  def body(x_ref, w_ref, b_ref, o0_ref, o1_ref):
    y = jnp.dot(x_ref[...], w_ref[...],
                preferred_element_type=_f32) + b_ref[...][None, :]
    o0_ref[...] = y[:, :HALF]
    o1_ref[...] = y[:, HALF:]

  return pl.pallas_call(
      body,
      grid=(np_rows // _BLK,),
      in_specs=[
          pl.BlockSpec((_BLK, in_dim), lambda i: (i, 0)),
          pl.BlockSpec((in_dim, 2 * HALF), lambda i: (0, 0)),
          pl.BlockSpec((2 * HALF,), lambda i: (0,)),
      ],
      out_specs=[pl.BlockSpec((_BLK, HALF), lambda i: (i, 0)),
                 pl.BlockSpec((_BLK, HALF), lambda i: (i, 0))],
      out_shape=[jax.ShapeDtypeStruct((np_rows, HALF), _f32),
                 jax.ShapeDtypeStruct((np_rows, HALF), _f32)],
  )


def _layer_norm_rows(y, g, b):
  m = jnp.mean(y, axis=-1, keepdims=True)
  v = jnp.mean((y - m) * (y - m), axis=-1, keepdims=True)
  return (y - m) / jnp.sqrt(v + 1e-5) * g[None, :] + b[None, :]


def _tc_update(np_rows, with_ctx):
  def body(*refs):
    if with_ctx:
      (h0_ref, h1_ref, a0_ref, a1_ref, inv_ref, ctx_ref, w_ref, b_ref,
       g_ref, bl_ref, o0_ref, o1_ref) = refs
    else:
      (h0_ref, h1_ref, a0_ref, a1_ref, inv_ref, w_ref, b_ref,
       g_ref, bl_ref, o0_ref, o1_ref) = refs
    inv = inv_ref[...][:, None]
    x0 = h0_ref[...] + a0_ref[0] * inv
    x1 = h1_ref[...] + a1_ref[0] * inv
    w = w_ref[...]
    y = (jnp.dot(x0, w[:HALF, :], preferred_element_type=_f32)
         + jnp.dot(x1, w[HALF:, :], preferred_element_type=_f32)
         + b_ref[...][None, :])
    if with_ctx:
      y = y + ctx_ref[...]
    y = _layer_norm_rows(y, g_ref[...], bl_ref[...])
    y = jnp.maximum(y, 0.0)
    o0_ref[...] = y[:, :HALF]
    o1_ref[...] = y[:, HALF:]

  in_specs = [
      pl.BlockSpec((_BLK, HALF), lambda i: (i, 0)),
      pl.BlockSpec((_BLK, HALF), lambda i: (i, 0)),
      pl.BlockSpec((1, _BLK, HALF), lambda i: (0, i, 0)),
      pl.BlockSpec((1, _BLK, HALF), lambda i: (1, i, 0)),
      pl.BlockSpec((_BLK,), lambda i: (i,)),
  ]
  if with_ctx:
    in_specs.append(pl.BlockSpec((_BLK, 2 * HALF), lambda i: (i, 0)))
  in_specs += [
      pl.BlockSpec((2 * HALF, 2 * HALF), lambda i: (0, 0)),
      pl.BlockSpec((2 * HALF,), lambda i: (0,)),
      pl.BlockSpec((2 * HALF,), lambda i: (0,)),
      pl.BlockSpec((2 * HALF,), lambda i: (0,)),
  ]
  return pl.pallas_call(
      body,
      grid=(np_rows // _BLK,),
      in_specs=in_specs,
      out_specs=[pl.BlockSpec((_BLK, HALF), lambda i: (i, 0)),
                 pl.BlockSpec((_BLK, HALF), lambda i: (i, 0))],
      out_shape=[jax.ShapeDtypeStruct((np_rows, HALF), _f32),
                 jax.ShapeDtypeStruct((np_rows, HALF), _f32)],
  )


def _tc_pool_cond(pool_rows, g_count, td):
  h = 2 * HALF

  def body(p0_ref, p1_ref, c0_ref, c1_ref, t_ref, fw_ref, fb_ref,
           tw1_ref, tb1_ref, tw2_ref, tb2_ref, cw1_ref, cb1_ref,
           cw2_ref, cb2_ref, o_ref):
    pool = jnp.concatenate([p0_ref[0], p1_ref[0]], axis=1)
    cnt = (c0_ref[...] + c1_ref[...])[:, None]
    mean = pool / jnp.maximum(cnt, 1.0)
    fo = jnp.dot(mean, fw_ref[...],
                 preferred_element_type=_f32) + fb_ref[...][None, :]
    left = fo[:g_count, :]
    right = fo[g_count:2 * g_count, :]
    half = td // 2
    i = lax.broadcasted_iota(jnp.int32, (g_count, half), 1).astype(_f32)
    freqs = jnp.exp((-math.log(10000.0) / half) * i)
    a = t_ref[...][:, None] * freqs
    te = jnp.concatenate([jnp.sin(a), jnp.cos(a)], axis=1)
    th = jnp.dot(te, tw1_ref[...],
                 preferred_element_type=_f32) + tb1_ref[...][None, :]
    th = th * jax.nn.sigmoid(th)
    th = jnp.dot(th, tw2_ref[...],
                 preferred_element_type=_f32) + tb2_ref[...][None, :]
    ci = jnp.concatenate([left, right, th], axis=1)
    gc = jnp.dot(ci, cw1_ref[...],
                 preferred_element_type=_f32) + cb1_ref[...][None, :]
    gc = gc * jax.nn.sigmoid(gc)
    gc = jnp.dot(gc, cw2_ref[...],
                 preferred_element_type=_f32) + cb2_ref[...][None, :]
    o_ref[...] = gc

  full = lambda *shape: pl.BlockSpec(shape, lambda: tuple(0 for _ in shape))
  return pl.pallas_call(
      body,
      in_specs=[
          full(1, pool_rows, HALF), full(1, pool_rows, HALF),
          full(pool_rows), full(pool_rows),
          full(g_count),
          full(h, h), full(h),
          full(td, h), full(h), full(h, h), full(h),
          full(3 * h, h), full(h), full(h, h), full(h),
      ],
      out_specs=full(g_count, h),
      out_shape=jax.ShapeDtypeStruct((g_count, h), _f32),
  )


def _tc_linker_in(np_rows, in_dim):
  def body(x_ref, nt_ref, ctx_ref, w_ref, b_ref, o0_ref, o1_ref):
    nt = nt_ref[...]
    ntc = jnp.clip(nt, 0, 2)
    w = w_ref[...]
    y = jnp.dot(x_ref[...], w[:in_dim, :],
                preferred_element_type=_f32) + b_ref[...][None, :]
    for kcls in range(3):
      y = y + (ntc == kcls).astype(_f32)[:, None] * w[in_dim + kcls][None, :]
    y = y + (nt > 0).astype(_f32)[:, None] * w[in_dim + 3][None, :]
    y = y + ctx_ref[...]
    o0_ref[...] = y[:, :HALF]
    o1_ref[...] = y[:, HALF:]

  return pl.pallas_call(
      body,
      grid=(np_rows // _BLK,),
      in_specs=[
          pl.BlockSpec((_BLK, in_dim), lambda i: (i, 0)),
          pl.BlockSpec((_BLK,), lambda i: (i,)),
          pl.BlockSpec((_BLK, 2 * HALF), lambda i: (i, 0)),
          pl.BlockSpec((in_dim + 4, 2 * HALF), lambda i: (0, 0)),
          pl.BlockSpec((2 * HALF,), lambda i: (0,)),
      ],
      out_specs=[pl.BlockSpec((_BLK, HALF), lambda i: (i, 0)),
                 pl.BlockSpec((_BLK, HALF), lambda i: (i, 0))],
      out_shape=[jax.ShapeDtypeStruct((np_rows, HALF), _f32),
                 jax.ShapeDtypeStruct((np_rows, HALF), _f32)],
  )


def _tc_out(np_rows, out_dim):
  def body(h0_ref, h1_ref, w_ref, b_ref, o_ref):
    w = w_ref[...]
    o_ref[...] = (jnp.dot(h0_ref[...], w[:HALF, :],
                          preferred_element_type=_f32)
                  + jnp.dot(h1_ref[...], w[HALF:, :],
                            preferred_element_type=_f32)
                  + b_ref[...][None, :])

  return pl.pallas_call(
      body,
      grid=(np_rows // _BLK,),
      in_specs=[
          pl.BlockSpec((_BLK, HALF), lambda i: (i, 0)),
          pl.BlockSpec((_BLK, HALF), lambda i: (i, 0)),
          pl.BlockSpec((2 * HALF, out_dim), lambda i: (0, 0)),
          pl.BlockSpec((out_dim,), lambda i: (0,)),
      ],
      out_specs=pl.BlockSpec((_BLK, out_dim), lambda i: (i, 0)),
      out_shape=jax.ShapeDtypeStruct((np_rows, out_dim), _f32),
  )


def _pad_nodes_2d(a, np_rows, fill=0.0):
  return jnp.pad(a, ((0, np_rows - a.shape[0]), (0, 0)),
                 constant_values=fill)


def _pad_ids(ids, np_rows, fill):
  return jnp.pad(ids.astype(jnp.int32), (0, np_rows - ids.shape[0]),
                 constant_values=fill)


def _prep_edges(src, dst, ep, dummy_dst):
  e = src.shape[0]
  src = jnp.pad(src.astype(jnp.int32), (0, ep - e), constant_values=0)
  dst = jnp.pad(dst.astype(jnp.int32), (0, ep - e),
                constant_values=dummy_dst)
  return src.reshape(ep // CH, CH), dst.reshape(ep // CH, CH)


def kernel(x, t, linker_batch, linker_graph_ptr, linker_node_type,
           linker_edge_index, left_x, left_edge_index, left_batch,
           right_x, right_edge_index, right_batch, params):
  G = int(linker_graph_ptr.shape[0]) - 1
  N = x.shape[1]
  IN = x.shape[2]
  NF = left_x.shape[0]
  E = linker_edge_index.shape[1]
  EF = left_edge_index.shape[1]
  TD = params['time_W1'].shape[0]

  Np = _ceil_to(max(N, 2 * NF), NC * NS * CH)
  e_unit = NS * CH * 2 * MACRO
  Ep = _ceil_to(E, e_unit)
  EFp = _ceil_to(2 * EF, e_unit)
  PoolR = _ceil_to(2 * G + 1, CH)

  p = params
  fp = p['frag']

  xL = _pad_nodes_2d(x[0], Np)
  ntL = _pad_ids(linker_node_type, Np, 0)
  batL = _pad_ids(linker_batch, Np, 0).reshape(Np // CH, CH)
  srcL, dstL = _prep_edges(linker_edge_index[0], linker_edge_index[1],
                           Ep, Np)

  xF = _pad_nodes_2d(jnp.concatenate([left_x, right_x], axis=0), Np)
  srcF = jnp.concatenate([left_edge_index[0],
                          right_edge_index[0] + NF], axis=0)
  dstF = jnp.concatenate([left_edge_index[1],
                          right_edge_index[1] + NF], axis=0)
  srcF, dstF = _prep_edges(srcF, dstF, EFp, Np)
  bF = jnp.concatenate([left_batch, right_batch + G], axis=0)
  bF = _pad_ids(bF, Np, 2 * G).reshape(Np // CH, CH)

  degL = _sc_hist(Np, Ep // CH)(dstL)
  degF = _sc_hist(Np, EFp // CH)(dstF)
  cntF = _sc_hist(PoolR, Np // CH)(bF)
  invL = _tc_invdeg(Np)(degL[0], degL[1])
  invF = _tc_invdeg(Np)(degF[0], degF[1])

  agg_f = _sc_agg(Np, EFp // CH)
  upd_f = _tc_update(Np, with_ctx=False)
  h0, h1 = _tc_frag_in(Np, IN)(xF, fp['in_W'], fp['in_b'])
  for i in range(len(fp['conv_W'])):
    hh = jnp.stack([h0, h1], axis=0)
    agg = agg_f(hh, srcF, dstF)
    h0, h1 = upd_f(h0, h1, agg, agg, invF, fp['conv_W'][i], fp['conv_b'][i],
                   fp['ln_g'][i], fp['ln_b'][i])

  pooled = _sc_pool(Np, PoolR)(jnp.stack([h0, h1], axis=0), bF)
  graph_ctx = _tc_pool_cond(PoolR, G, TD)(
      pooled[0:1], pooled[1:2], cntF[0], cntF[1], t,
      fp['out_W'], fp['out_b'],
      p['time_W1'], p['time_b1'], p['time_W2'], p['time_b2'],
      p['cond_W1'], p['cond_b1'], p['cond_W2'], p['cond_b2'])

  ctx = _sc_gather_rows(Np, 2 * HALF)(graph_ctx, batL)
  h0, h1 = _tc_linker_in(Np, IN)(xL, ntL, ctx, p['in_W'], p['in_b'])
  agg_l = _sc_agg(Np, Ep // CH)
  upd_l = _tc_update(Np, with_ctx=True)
  for i in range(len(p['conv_W'])):
    hh = jnp.stack([h0, h1], axis=0)
    agg = agg_l(hh, srcL, dstL)
    h0, h1 = upd_l(h0, h1, agg, agg, invL, ctx, p['conv_W'][i],
                   p['conv_b'][i], p['ln_g'][i], p['ln_b'][i])

  out = _tc_out(Np, p['out_W'].shape[1])(h0, h1, p['out_W'], p['out_b'])
  return out[:N][None]


# speedup vs baseline: 1.2599x; 1.2599x over previous
import functools
import math

import jax
import jax.numpy as jnp
from jax import lax
from jax.experimental import pallas as pl
from jax.experimental.pallas import tpu as pltpu
from jax.experimental.pallas import tpu_sc as plsc

NC = 2
NS = 16
LANES = 16
CH = 128
MACRO = 4
HALF = 32

_f32 = jnp.float32


def _ceil_to(x, m):
  return (x + m - 1) // m * m


def _zero_fill(buf_ref, nrows, ncols):
  per_row = ncols // LANES

  def body(i, _):
    r = i // per_row
    c = (i % per_row) * LANES
    buf_ref[r, pl.ds(c, LANES)] = jnp.zeros((LANES,), _f32)
    return 0

  lax.fori_loop(0, nrows * per_row, body, 0)


def _const_fill(buf_ref, nrows, ncols, val):
  per_row = ncols // LANES

  def body(i, _):
    r = i // per_row
    c = (i % per_row) * LANES
    buf_ref[r, pl.ds(c, LANES)] = jnp.full((LANES,), val, _f32)
    return 0

  lax.fori_loop(0, nrows * per_row, body, 0)


def _sc_hist(np_rows, n_chunks):
  mesh = plsc.VectorSubcoreMesh(core_axis_name="c", subcore_axis_name="s",
                                num_cores=NC, num_subcores=NS)
  acc_rows = np_rows + CH
  out_chunks = np_rows // CH

  @functools.partial(
      pl.kernel,
      mesh=mesh,
      compiler_params=pltpu.CompilerParams(use_tc_tiling_on_sc=False),
      out_type=jax.ShapeDtypeStruct((NC, np_rows), _f32),
      scratch_types=[
          pltpu.VMEM_SHARED((acc_rows,), _f32),
          pltpu.VMEM((CH,), jnp.int32),
          pltpu.VMEM((CH,), _f32),
          pltpu.VMEM((CH,), _f32),
      ],
  )
  def k(dst_hbm, out_hbm, acc, idx_v, ones_v, buf_v):
    c = lax.axis_index("c")
    s = lax.axis_index("s")

    def fill(i, _):
      buf_v[pl.ds(i * LANES, LANES)] = jnp.zeros((LANES,), _f32)
      ones_v[pl.ds(i * LANES, LANES)] = jnp.ones((LANES,), _f32)
      return 0

    lax.fori_loop(0, CH // LANES, fill, 0)

    nz = acc_rows // CH

    def zbody(i, _):
      pltpu.sync_copy(buf_v, acc.at[pl.ds((i * NS + s) * CH, CH)])
      return 0

    lax.fori_loop(0, nz // NS, zbody, 0)

    @pl.when(s < nz % NS)
    def _():
      pltpu.sync_copy(buf_v, acc.at[pl.ds(((nz // NS) * NS + s) * CH, CH)])

    plsc.subcore_barrier()

    half = n_chunks // NC

    def ebody(i, _):
      ch = c * half + i * NS + s
      pltpu.sync_copy(dst_hbm.at[ch], idx_v)
      pltpu.sync_copy(ones_v, acc.at[idx_v], add=True)
      return 0

    lax.fori_loop(0, half // NS, ebody, 0)

    @pl.when(s < half % NS)
    def _():
      ch = c * half + (half // NS) * NS + s
      pltpu.sync_copy(dst_hbm.at[ch], idx_v)
      pltpu.sync_copy(ones_v, acc.at[idx_v], add=True)

    plsc.subcore_barrier()

    def obody(i, _):
      r = (i * NS + s) * CH
      pltpu.sync_copy(acc.at[pl.ds(r, CH)], buf_v)
      pltpu.sync_copy(buf_v, out_hbm.at[c, pl.ds(r, CH)])
      return 0

    lax.fori_loop(0, out_chunks // NS, obody, 0)

    @pl.when(s < out_chunks % NS)
    def _():
      r = ((out_chunks // NS) * NS + s) * CH
      pltpu.sync_copy(acc.at[pl.ds(r, CH)], buf_v)
      pltpu.sync_copy(buf_v, out_hbm.at[c, pl.ds(r, CH)])

  return k


def _sc_agg(np_rows, n_chunks):
  mesh = plsc.VectorSubcoreMesh(core_axis_name="c", subcore_axis_name="s",
                                num_cores=NC, num_subcores=NS)
  acc_rows = np_rows + CH
  out_chunks = np_rows // CH
  n_pair = n_chunks // (2 * MACRO)

  @functools.partial(
      pl.kernel,
      mesh=mesh,
      compiler_params=pltpu.CompilerParams(use_tc_tiling_on_sc=False),
      out_type=jax.ShapeDtypeStruct((NC, np_rows, HALF), _f32),
      scratch_types=[
          pltpu.VMEM_SHARED((acc_rows, HALF), _f32),
          pltpu.VMEM((MACRO, CH), jnp.int32),
          pltpu.VMEM((MACRO, CH), jnp.int32),
          pltpu.VMEM((MACRO, CH), jnp.int32),
          pltpu.VMEM((MACRO, CH), jnp.int32),
          pltpu.VMEM((MACRO * CH, HALF), _f32),
          pltpu.SemaphoreType.DMA,
          pltpu.SemaphoreType.DMA,
      ],
  )
  def k(h_hbm, src_hbm, dst_hbm, out_hbm, acc, src_a, dst_a, src_b, dst_b,
        rows_v, gsem, ssem):
    c = lax.axis_index("c")
    s = lax.axis_index("s")
    buf_v = rows_v.at[pl.ds(0, CH)]
    _zero_fill(rows_v, CH, HALF)

    nz = acc_rows // CH

    def zbody(i, _):
      pltpu.sync_copy(buf_v, acc.at[pl.ds((i * NS + s) * CH, CH)])
      return 0

    lax.fori_loop(0, nz // NS, zbody, 0)

    @pl.when(s < nz % NS)
    def _():
      pltpu.sync_copy(buf_v, acc.at[pl.ds(((nz // NS) * NS + s) * CH, CH)])

    plsc.subcore_barrier()

    n_tile_macros = n_chunks // MACRO // NS
    max_m = (n_chunks // MACRO - 1) * MACRO

    def run_macro(src_i, dst_i):
      ga = [pltpu.async_copy(h_hbm.at[c].at[src_i.at[j]],
                             rows_v.at[pl.ds(j * CH, CH)], gsem)
            for j in range(MACRO)]
      return ga

    def load_idx(k, src_i, dst_i):
      m = jnp.minimum((k * NS + s) * MACRO, max_m)
      pltpu.sync_copy(src_hbm.at[pl.ds(m, MACRO)], src_i)
      pltpu.sync_copy(dst_hbm.at[pl.ds(m, MACRO)], dst_i)

    def drain_scatter(dst_i):
      sa = [pltpu.async_copy(rows_v.at[pl.ds(j * CH, CH)],
                             acc.at[dst_i.at[j]], ssem, add=True)
            for j in range(MACRO)]
      for d in sa:
        d.wait()

    load_idx(0, src_a, dst_a)

    def ebody(i, _):
      ga = run_macro(src_a, dst_a)
      load_idx(2 * i + 1, src_b, dst_b)
      for d in ga:
        d.wait()
      drain_scatter(dst_a)
      gb = run_macro(src_b, dst_b)
      load_idx(2 * i + 2, src_a, dst_a)
      for d in gb:
        d.wait()
      drain_scatter(dst_b)
      return 0

    lax.fori_loop(0, n_tile_macros // 2, ebody, 0)

    plsc.subcore_barrier()

    def obody(i, _):
      r = (i * NS + s) * CH
      pltpu.sync_copy(acc.at[pl.ds(r, CH)], buf_v)
      pltpu.sync_copy(buf_v, out_hbm.at[c, pl.ds(r, CH)])
      return 0

    lax.fori_loop(0, out_chunks // NS, obody, 0)

    @pl.when(s < out_chunks % NS)
    def _():
      r = ((out_chunks // NS) * NS + s) * CH
      pltpu.sync_copy(acc.at[pl.ds(r, CH)], buf_v)
      pltpu.sync_copy(buf_v, out_hbm.at[c, pl.ds(r, CH)])

  return k


def _sc_pool(np_rows, pool_rows):
  mesh = plsc.VectorSubcoreMesh(core_axis_name="c", subcore_axis_name="s",
                                num_cores=NC, num_subcores=NS)
  n_chunks = np_rows // CH
  out_chunks = pool_rows // CH

  @functools.partial(
      pl.kernel,
      mesh=mesh,
      compiler_params=pltpu.CompilerParams(use_tc_tiling_on_sc=False),
      out_type=jax.ShapeDtypeStruct((NC, pool_rows, HALF), _f32),
      scratch_types=[
          pltpu.VMEM_SHARED((pool_rows, HALF), _f32),
          pltpu.VMEM((CH,), jnp.int32),
          pltpu.VMEM((CH, HALF), _f32),
          pltpu.VMEM((CH, HALF), _f32),
      ],
  )
  def k(h_hbm, b_hbm, out_hbm, acc, idx_v, vals_v, buf_v):
    c = lax.axis_index("c")
    s = lax.axis_index("s")
    _zero_fill(buf_v, CH, HALF)

    @pl.when(s < out_chunks)
    def _():
      pltpu.sync_copy(buf_v, acc.at[pl.ds(s * CH, CH)])

    plsc.subcore_barrier()

    def ebody(i, _):
      ch = i * NS + s
      pltpu.sync_copy(b_hbm.at[ch], idx_v)
      pltpu.sync_copy(h_hbm.at[c, pl.ds(ch * CH, CH)], vals_v)
      pltpu.sync_copy(vals_v, acc.at[idx_v], add=True)
      return 0

    lax.fori_loop(0, n_chunks // NS, ebody, 0)

    @pl.when(s < n_chunks % NS)
    def _():
      ch = (n_chunks // NS) * NS + s
      pltpu.sync_copy(b_hbm.at[ch], idx_v)
      pltpu.sync_copy(h_hbm.at[c, pl.ds(ch * CH, CH)], vals_v)
      pltpu.sync_copy(vals_v, acc.at[idx_v], add=True)

    plsc.subcore_barrier()

    @pl.when(s < out_chunks)
    def _():
      pltpu.sync_copy(acc.at[pl.ds(s * CH, CH)], buf_v)
      pltpu.sync_copy(buf_v, out_hbm.at[c, pl.ds(s * CH, CH)])

  return k


def _sc_gather_rows(np_rows, d):
  mesh = plsc.VectorSubcoreMesh(core_axis_name="c", subcore_axis_name="s",
                                num_cores=NC, num_subcores=NS)
  n_chunks = np_rows // CH

  @functools.partial(
      pl.kernel,
      mesh=mesh,
      compiler_params=pltpu.CompilerParams(use_tc_tiling_on_sc=False),
      out_type=jax.ShapeDtypeStruct((np_rows, d), _f32),
      scratch_types=[
          pltpu.VMEM((CH,), jnp.int32),
          pltpu.VMEM((CH, d), _f32),
          pltpu.SemaphoreType.DMA,
      ],
  )
  def k(tab_hbm, idx_hbm, out_hbm, idx_v, rows_v, sem):
    c = lax.axis_index("c")
    s = lax.axis_index("s")
    w = s * NC + c
    nw = NC * NS

    def body(i, _):
      ch = i * nw + w
      pltpu.sync_copy(idx_hbm.at[ch], idx_v)
      pltpu.async_copy(tab_hbm.at[idx_v], rows_v, sem).wait()
      pltpu.sync_copy(rows_v, out_hbm.at[pl.ds(ch * CH, CH)])
      return 0

    lax.fori_loop(0, n_chunks // nw, body, 0)

  return k


_BLK = 256


def _tc_invdeg(np_rows):
  blk = 1024

  def body(d0_ref, d1_ref, o_ref):
    d = d0_ref[...] + d1_ref[...]
    o_ref[...] = 1.0 / jnp.maximum(d, 1.0)

  return pl.pallas_call(
      body,
      grid=(np_rows // blk,),
      in_specs=[pl.BlockSpec((blk,), lambda i: (i,)),
                pl.BlockSpec((blk,), lambda i: (i,))],
      out_specs=pl.BlockSpec((blk,), lambda i: (i,)),
      out_shape=jax.ShapeDtypeStruct((np_rows,), _f32),
  )


def _tc_frag_in(np_rows, in_dim):
  def body(x_ref, w_ref, b_ref, o_ref):
    y = jnp.dot(x_ref[...], w_ref[...],
                preferred_element_type=_f32) + b_ref[...][None, :]
    o_ref[0] = y[:, :HALF]
    o_ref[1] = y[:, HALF:]

  return pl.pallas_call(
      body,
      grid=(np_rows // _BLK,),
      in_specs=[
          pl.BlockSpec((_BLK, in_dim), lambda i: (i, 0)),
          pl.BlockSpec((in_dim, 2 * HALF), lambda i: (0, 0)),
          pl.BlockSpec((2 * HALF,), lambda i: (0,)),
      ],
      out_specs=pl.BlockSpec((NC, _BLK, HALF), lambda i: (0, i, 0)),
      out_shape=jax.ShapeDtypeStruct((NC, np_rows, HALF), _f32),
  )


def _layer_norm_rows(y, g, b):
  m = jnp.mean(y, axis=-1, keepdims=True)
  v = jnp.mean((y - m) * (y - m), axis=-1, keepdims=True)
  return (y - m) / jnp.sqrt(v + 1e-5) * g[None, :] + b[None, :]


def _tc_update(np_rows, with_ctx):
  def body(*refs):
    if with_ctx:
      (h_ref, a_ref, inv_ref, ctx_ref, w_ref, b_ref,
       g_ref, bl_ref, o_ref) = refs
    else:
      (h_ref, a_ref, inv_ref, w_ref, b_ref, g_ref, bl_ref, o_ref) = refs
    inv = inv_ref[...][:, None]
    x0 = h_ref[0] + a_ref[0] * inv
    x1 = h_ref[1] + a_ref[1] * inv
    w = w_ref[...]
    y = (jnp.dot(x0, w[:HALF, :], preferred_element_type=_f32)
         + jnp.dot(x1, w[HALF:, :], preferred_element_type=_f32)
         + b_ref[...][None, :])
    if with_ctx:
      y = y + ctx_ref[...]
    y = _layer_norm_rows(y, g_ref[...], bl_ref[...])
    y = jnp.maximum(y, 0.0)
    o_ref[0] = y[:, :HALF]
    o_ref[1] = y[:, HALF:]

  in_specs = [
      pl.BlockSpec((NC, _BLK, HALF), lambda i: (0, i, 0)),
      pl.BlockSpec((NC, _BLK, HALF), lambda i: (0, i, 0)),
      pl.BlockSpec((_BLK,), lambda i: (i,)),
  ]
  if with_ctx:
    in_specs.append(pl.BlockSpec((_BLK, 2 * HALF), lambda i: (i, 0)))
  in_specs += [
      pl.BlockSpec((2 * HALF, 2 * HALF), lambda i: (0, 0)),
      pl.BlockSpec((2 * HALF,), lambda i: (0,)),
      pl.BlockSpec((2 * HALF,), lambda i: (0,)),
      pl.BlockSpec((2 * HALF,), lambda i: (0,)),
  ]
  return pl.pallas_call(
      body,
      grid=(np_rows // _BLK,),
      in_specs=in_specs,
      out_specs=pl.BlockSpec((NC, _BLK, HALF), lambda i: (0, i, 0)),
      out_shape=jax.ShapeDtypeStruct((NC, np_rows, HALF), _f32),
  )


def _tc_pool_cond(pool_rows, g_count, td):
  h = 2 * HALF

  def body(p0_ref, p1_ref, c0_ref, c1_ref, t_ref, fw_ref, fb_ref,
           tw1_ref, tb1_ref, tw2_ref, tb2_ref, cw1_ref, cb1_ref,
           cw2_ref, cb2_ref, o_ref):
    pool = jnp.concatenate([p0_ref[0], p1_ref[0]], axis=1)
    cnt = (c0_ref[...] + c1_ref[...])[:, None]
    mean = pool / jnp.maximum(cnt, 1.0)
    fo = jnp.dot(mean, fw_ref[...],
                 preferred_element_type=_f32) + fb_ref[...][None, :]
    left = fo[:g_count, :]
    right = fo[g_count:2 * g_count, :]
    half = td // 2
    i = lax.broadcasted_iota(jnp.int32, (g_count, half), 1).astype(_f32)
    freqs = jnp.exp((-math.log(10000.0) / half) * i)
    a = t_ref[...][:, None] * freqs
    te = jnp.concatenate([jnp.sin(a), jnp.cos(a)], axis=1)
    th = jnp.dot(te, tw1_ref[...],
                 preferred_element_type=_f32) + tb1_ref[...][None, :]
    th = th * jax.nn.sigmoid(th)
    th = jnp.dot(th, tw2_ref[...],
                 preferred_element_type=_f32) + tb2_ref[...][None, :]
    ci = jnp.concatenate([left, right, th], axis=1)
    gc = jnp.dot(ci, cw1_ref[...],
                 preferred_element_type=_f32) + cb1_ref[...][None, :]
    gc = gc * jax.nn.sigmoid(gc)
    gc = jnp.dot(gc, cw2_ref[...],
                 preferred_element_type=_f32) + cb2_ref[...][None, :]
    o_ref[...] = gc

  full = lambda *shape: pl.BlockSpec(shape, lambda: tuple(0 for _ in shape))
  return pl.pallas_call(
      body,
      in_specs=[
          full(1, pool_rows, HALF), full(1, pool_rows, HALF),
          full(pool_rows), full(pool_rows),
          full(g_count),
          full(h, h), full(h),
          full(td, h), full(h), full(h, h), full(h),
          full(3 * h, h), full(h), full(h, h), full(h),
      ],
      out_specs=full(g_count, h),
      out_shape=jax.ShapeDtypeStruct((g_count, h), _f32),
  )


def _tc_linker_in(np_rows, in_dim):
  def body(x_ref, nt_ref, ctx_ref, w_ref, b_ref, o_ref):
    nt = nt_ref[...]
    ntc = jnp.clip(nt, 0, 2)
    w = w_ref[...]
    y = jnp.dot(x_ref[...], w[:in_dim, :],
                preferred_element_type=_f32) + b_ref[...][None, :]
    for kcls in range(3):
      y = y + (ntc == kcls).astype(_f32)[:, None] * w[in_dim + kcls][None, :]
    y = y + (nt > 0).astype(_f32)[:, None] * w[in_dim + 3][None, :]
    y = y + ctx_ref[...]
    o_ref[0] = y[:, :HALF]
    o_ref[1] = y[:, HALF:]

  return pl.pallas_call(
      body,
      grid=(np_rows // _BLK,),
      in_specs=[
          pl.BlockSpec((_BLK, in_dim), lambda i: (i, 0)),
          pl.BlockSpec((_BLK,), lambda i: (i,)),
          pl.BlockSpec((_BLK, 2 * HALF), lambda i: (i, 0)),
          pl.BlockSpec((in_dim + 4, 2 * HALF), lambda i: (0, 0)),
          pl.BlockSpec((2 * HALF,), lambda i: (0,)),
      ],
      out_specs=pl.BlockSpec((NC, _BLK, HALF), lambda i: (0, i, 0)),
      out_shape=jax.ShapeDtypeStruct((NC, np_rows, HALF), _f32),
  )


def _tc_out(np_rows, out_dim):
  def body(h_ref, w_ref, b_ref, o_ref):
    w = w_ref[...]
    o_ref[...] = (jnp.dot(h_ref[0], w[:HALF, :],
                          preferred_element_type=_f32)
                  + jnp.dot(h_ref[1], w[HALF:, :],
                            preferred_element_type=_f32)
                  + b_ref[...][None, :])

  return pl.pallas_call(
      body,
      grid=(np_rows // _BLK,),
      in_specs=[
          pl.BlockSpec((NC, _BLK, HALF), lambda i: (0, i, 0)),
          pl.BlockSpec((2 * HALF, out_dim), lambda i: (0, 0)),
          pl.BlockSpec((out_dim,), lambda i: (0,)),
      ],
      out_specs=pl.BlockSpec((_BLK, out_dim), lambda i: (i, 0)),
      out_shape=jax.ShapeDtypeStruct((np_rows, out_dim), _f32),
  )


def _pad_nodes_2d(a, np_rows, fill=0.0):
  return jnp.pad(a, ((0, np_rows - a.shape[0]), (0, 0)),
                 constant_values=fill)


def _pad_ids(ids, np_rows, fill):
  return jnp.pad(ids.astype(jnp.int32), (0, np_rows - ids.shape[0]),
                 constant_values=fill)


def _prep_edges(src, dst, ep, dummy_dst):
  e = src.shape[0]
  src = jnp.pad(src.astype(jnp.int32), (0, ep - e), constant_values=0)
  dst = jnp.pad(dst.astype(jnp.int32), (0, ep - e),
                constant_values=dummy_dst)
  return src.reshape(ep // CH, CH), dst.reshape(ep // CH, CH)


def kernel(x, t, linker_batch, linker_graph_ptr, linker_node_type,
           linker_edge_index, left_x, left_edge_index, left_batch,
           right_x, right_edge_index, right_batch, params):
  G = int(linker_graph_ptr.shape[0]) - 1
  N = x.shape[1]
  IN = x.shape[2]
  NF = left_x.shape[0]
  E = linker_edge_index.shape[1]
  EF = left_edge_index.shape[1]
  TD = params['time_W1'].shape[0]

  Np = _ceil_to(max(N, 2 * NF), NC * NS * CH)
  e_unit = NS * CH * 2 * MACRO
  Ep = _ceil_to(E, e_unit)
  EFp = _ceil_to(2 * EF, e_unit)
  PoolR = _ceil_to(2 * G + 1, CH)

  p = params
  fp = p['frag']

  xL = _pad_nodes_2d(x[0], Np)
  ntL = _pad_ids(linker_node_type, Np, 0)
  batL = _pad_ids(linker_batch, Np, 0).reshape(Np // CH, CH)
  srcL, dstL = _prep_edges(linker_edge_index[0], linker_edge_index[1],
                           Ep, Np)

  xF = _pad_nodes_2d(jnp.concatenate([left_x, right_x], axis=0), Np)
  srcF = jnp.concatenate([left_edge_index[0],
                          right_edge_index[0] + NF], axis=0)
  dstF = jnp.concatenate([left_edge_index[1],
                          right_edge_index[1] + NF], axis=0)
  srcF, dstF = _prep_edges(srcF, dstF, EFp, Np)
  bF = jnp.concatenate([left_batch, right_batch + G], axis=0)
  bF = _pad_ids(bF, Np, 2 * G).reshape(Np // CH, CH)

  degL = _sc_hist(Np, Ep // CH)(dstL)
  degF = _sc_hist(Np, EFp // CH)(dstF)
  cntF = _sc_hist(PoolR, Np // CH)(bF)
  invL = _tc_invdeg(Np)(degL[0], degL[1])
  invF = _tc_invdeg(Np)(degF[0], degF[1])

  agg_f = _sc_agg(Np, EFp // CH)
  upd_f = _tc_update(Np, with_ctx=False)
  hs = _tc_frag_in(Np, IN)(xF, fp['in_W'], fp['in_b'])
  for i in range(len(fp['conv_W'])):
    agg = agg_f(hs, srcF, dstF)
    hs = upd_f(hs, agg, invF, fp['conv_W'][i], fp['conv_b'][i],
               fp['ln_g'][i], fp['ln_b'][i])

  pooled = _sc_pool(Np, PoolR)(hs, bF)
  graph_ctx = _tc_pool_cond(PoolR, G, TD)(
      pooled[0:1], pooled[1:2], cntF[0], cntF[1], t,
      fp['out_W'], fp['out_b'],
      p['time_W1'], p['time_b1'], p['time_W2'], p['time_b2'],
      p['cond_W1'], p['cond_b1'], p['cond_W2'], p['cond_b2'])

  ctx = _sc_gather_rows(Np, 2 * HALF)(graph_ctx, batL)
  hs = _tc_linker_in(Np, IN)(xL, ntL, ctx, p['in_W'], p['in_b'])
  agg_l = _sc_agg(Np, Ep // CH)
  upd_l = _tc_update(Np, with_ctx=True)
  for i in range(len(p['conv_W'])):
    agg = agg_l(hs, srcL, dstL)
    hs = upd_l(hs, agg, invL, ctx, p['conv_W'][i],
               p['conv_b'][i], p['ln_g'][i], p['ln_b'][i])

  out = _tc_out(Np, p['out_W'].shape[1])(hs, p['out_W'], p['out_b'])
  return out[:N][None]


# speedup vs baseline: 1.7111x; 1.3581x over previous
import functools
import math

import jax
import jax.numpy as jnp
from jax import lax
from jax.experimental import pallas as pl
from jax.experimental.pallas import tpu as pltpu
from jax.experimental.pallas import tpu_sc as plsc

NC = 2
NS = 16
LANES = 16
CH = 128
MACRO = 4
HALF = 32

_f32 = jnp.float32


def _ceil_to(x, m):
  return (x + m - 1) // m * m


def _zero_fill(buf_ref, nrows, ncols):
  per_row = ncols // LANES

  def body(i, _):
    r = i // per_row
    c = (i % per_row) * LANES
    buf_ref[r, pl.ds(c, LANES)] = jnp.zeros((LANES,), _f32)
    return 0

  lax.fori_loop(0, nrows * per_row, body, 0)


def _const_fill(buf_ref, nrows, ncols, val):
  per_row = ncols // LANES

  def body(i, _):
    r = i // per_row
    c = (i % per_row) * LANES
    buf_ref[r, pl.ds(c, LANES)] = jnp.full((LANES,), val, _f32)
    return 0

  lax.fori_loop(0, nrows * per_row, body, 0)


def _sc_hist(np_rows, n_chunks):
  mesh = plsc.VectorSubcoreMesh(core_axis_name="c", subcore_axis_name="s",
                                num_cores=NC, num_subcores=NS)
  acc_rows = np_rows + CH
  out_chunks = np_rows // CH

  @functools.partial(
      pl.kernel,
      mesh=mesh,
      compiler_params=pltpu.CompilerParams(use_tc_tiling_on_sc=False),
      out_type=jax.ShapeDtypeStruct((NC, np_rows), _f32),
      scratch_types=[
          pltpu.VMEM_SHARED((acc_rows,), _f32),
          pltpu.VMEM((CH,), jnp.int32),
          pltpu.VMEM((CH,), _f32),
          pltpu.VMEM((CH,), _f32),
      ],
  )
  def k(dst_hbm, out_hbm, acc, idx_v, ones_v, buf_v):
    c = lax.axis_index("c")
    s = lax.axis_index("s")

    def fill(i, _):
      buf_v[pl.ds(i * LANES, LANES)] = jnp.zeros((LANES,), _f32)
      ones_v[pl.ds(i * LANES, LANES)] = jnp.ones((LANES,), _f32)
      return 0

    lax.fori_loop(0, CH // LANES, fill, 0)

    nz = acc_rows // CH

    def zbody(i, _):
      pltpu.sync_copy(buf_v, acc.at[pl.ds((i * NS + s) * CH, CH)])
      return 0

    lax.fori_loop(0, nz // NS, zbody, 0)

    @pl.when(s < nz % NS)
    def _():
      pltpu.sync_copy(buf_v, acc.at[pl.ds(((nz // NS) * NS + s) * CH, CH)])

    plsc.subcore_barrier()

    half = n_chunks // NC

    def ebody(i, _):
      ch = c * half + i * NS + s
      pltpu.sync_copy(dst_hbm.at[ch], idx_v)
      pltpu.sync_copy(ones_v, acc.at[idx_v], add=True)
      return 0

    lax.fori_loop(0, half // NS, ebody, 0)

    @pl.when(s < half % NS)
    def _():
      ch = c * half + (half // NS) * NS + s
      pltpu.sync_copy(dst_hbm.at[ch], idx_v)
      pltpu.sync_copy(ones_v, acc.at[idx_v], add=True)

    plsc.subcore_barrier()

    def obody(i, _):
      r = (i * NS + s) * CH
      pltpu.sync_copy(acc.at[pl.ds(r, CH)], buf_v)
      pltpu.sync_copy(buf_v, out_hbm.at[c, pl.ds(r, CH)])
      return 0

    lax.fori_loop(0, out_chunks // NS, obody, 0)

    @pl.when(s < out_chunks % NS)
    def _():
      r = ((out_chunks // NS) * NS + s) * CH
      pltpu.sync_copy(acc.at[pl.ds(r, CH)], buf_v)
      pltpu.sync_copy(buf_v, out_hbm.at[c, pl.ds(r, CH)])

  return k


def _sc_agg(np_rows, n_chunks):
  mesh = plsc.VectorSubcoreMesh(core_axis_name="c", subcore_axis_name="s",
                                num_cores=NC, num_subcores=NS)
  acc_rows = np_rows + CH
  out_chunks = np_rows // CH
  n_pair = n_chunks // (2 * MACRO)

  @functools.partial(
      pl.kernel,
      mesh=mesh,
      compiler_params=pltpu.CompilerParams(use_tc_tiling_on_sc=False),
      out_type=jax.ShapeDtypeStruct((np_rows, NC, HALF), _f32),
      scratch_types=[
          pltpu.VMEM_SHARED((acc_rows, HALF), _f32),
          pltpu.VMEM((MACRO, CH), jnp.int32),
          pltpu.VMEM((MACRO, CH), jnp.int32),
          pltpu.VMEM((MACRO, CH), jnp.int32),
          pltpu.VMEM((MACRO, CH), jnp.int32),
          pltpu.VMEM((MACRO * CH, HALF), _f32),
          pltpu.SemaphoreType.DMA,
          pltpu.SemaphoreType.DMA,
      ],
  )
  def k(h_hbm, src_hbm, dst_hbm, out_hbm, acc, src_a, dst_a, src_b, dst_b,
        rows_v, gsem, ssem):
    c = lax.axis_index("c")
    s = lax.axis_index("s")
    buf_v = rows_v.at[pl.ds(0, CH)]
    _zero_fill(rows_v, CH, HALF)

    nz = acc_rows // CH

    def zbody(i, _):
      pltpu.sync_copy(buf_v, acc.at[pl.ds((i * NS + s) * CH, CH)])
      return 0

    lax.fori_loop(0, nz // NS, zbody, 0)

    @pl.when(s < nz % NS)
    def _():
      pltpu.sync_copy(buf_v, acc.at[pl.ds(((nz // NS) * NS + s) * CH, CH)])

    plsc.subcore_barrier()

    n_tile_macros = n_chunks // MACRO // NS
    max_m = (n_chunks // MACRO - 1) * MACRO

    def run_macro(src_i):
      ga = [pltpu.async_copy(h_hbm.at[src_i.at[j]],
                             rows_v.at[pl.ds(j * CH, CH)], gsem)
            for j in range(MACRO)]
      return ga

    def load_idx(k, src_i, dst_i):
      m = jnp.minimum((k * NS + s) * MACRO, max_m)
      pltpu.sync_copy(src_hbm.at[pl.ds(m, MACRO)], src_i)
      pltpu.sync_copy(dst_hbm.at[pl.ds(m, MACRO)], dst_i)
      for j in range(MACRO):
        for t in range(CH // LANES):
          v = src_i[j, pl.ds(t * LANES, LANES)]
          src_i[j, pl.ds(t * LANES, LANES)] = v * 2 + c

    def drain_scatter(dst_i):
      sa = [pltpu.async_copy(rows_v.at[pl.ds(j * CH, CH)],
                             acc.at[dst_i.at[j]], ssem, add=True)
            for j in range(MACRO)]
      for d in sa:
        d.wait()

    load_idx(0, src_a, dst_a)

    def ebody(i, _):
      ga = run_macro(src_a)
      load_idx(2 * i + 1, src_b, dst_b)
      for d in ga:
        d.wait()
      drain_scatter(dst_a)
      gb = run_macro(src_b)
      load_idx(2 * i + 2, src_a, dst_a)
      for d in gb:
        d.wait()
      drain_scatter(dst_b)
      return 0

    lax.fori_loop(0, n_tile_macros // 2, ebody, 0)

    plsc.subcore_barrier()

    def obody(i, _):
      r = (i * NS + s) * CH
      pltpu.sync_copy(acc.at[pl.ds(r, CH)], buf_v)
      pltpu.sync_copy(buf_v, out_hbm.at[pl.ds(r, CH), c])
      return 0

    lax.fori_loop(0, out_chunks // NS, obody, 0)

    @pl.when(s < out_chunks % NS)
    def _():
      r = ((out_chunks // NS) * NS + s) * CH
      pltpu.sync_copy(acc.at[pl.ds(r, CH)], buf_v)
      pltpu.sync_copy(buf_v, out_hbm.at[pl.ds(r, CH), c])

  return k


def _sc_pool(np_rows, pool_rows):
  mesh = plsc.VectorSubcoreMesh(core_axis_name="c", subcore_axis_name="s",
                                num_cores=NC, num_subcores=NS)
  n_chunks = np_rows // CH
  out_chunks = pool_rows // CH

  @functools.partial(
      pl.kernel,
      mesh=mesh,
      compiler_params=pltpu.CompilerParams(use_tc_tiling_on_sc=False),
      out_type=jax.ShapeDtypeStruct((NC, pool_rows, HALF), _f32),
      scratch_types=[
          pltpu.VMEM_SHARED((pool_rows, HALF), _f32),
          pltpu.VMEM((CH,), jnp.int32),
          pltpu.VMEM((CH, HALF), _f32),
          pltpu.VMEM((CH, HALF), _f32),
      ],
  )
  def k(h_hbm, b_hbm, out_hbm, acc, idx_v, vals_v, buf_v):
    c = lax.axis_index("c")
    s = lax.axis_index("s")
    _zero_fill(buf_v, CH, HALF)

    @pl.when(s < out_chunks)
    def _():
      pltpu.sync_copy(buf_v, acc.at[pl.ds(s * CH, CH)])

    plsc.subcore_barrier()

    def ebody(i, _):
      ch = i * NS + s
      pltpu.sync_copy(b_hbm.at[ch], idx_v)
      pltpu.sync_copy(h_hbm.at[pl.ds(ch * CH, CH), c], vals_v)
      pltpu.sync_copy(vals_v, acc.at[idx_v], add=True)
      return 0

    lax.fori_loop(0, n_chunks // NS, ebody, 0)

    @pl.when(s < n_chunks % NS)
    def _():
      ch = (n_chunks // NS) * NS + s
      pltpu.sync_copy(b_hbm.at[ch], idx_v)
      pltpu.sync_copy(h_hbm.at[pl.ds(ch * CH, CH), c], vals_v)
      pltpu.sync_copy(vals_v, acc.at[idx_v], add=True)

    plsc.subcore_barrier()

    @pl.when(s < out_chunks)
    def _():
      pltpu.sync_copy(acc.at[pl.ds(s * CH, CH)], buf_v)
      pltpu.sync_copy(buf_v, out_hbm.at[c, pl.ds(s * CH, CH)])

  return k


def _sc_gather_rows(np_rows, d):
  mesh = plsc.VectorSubcoreMesh(core_axis_name="c", subcore_axis_name="s",
                                num_cores=NC, num_subcores=NS)
  n_chunks = np_rows // CH

  @functools.partial(
      pl.kernel,
      mesh=mesh,
      compiler_params=pltpu.CompilerParams(use_tc_tiling_on_sc=False),
      out_type=jax.ShapeDtypeStruct((np_rows, d), _f32),
      scratch_types=[
          pltpu.VMEM((CH,), jnp.int32),
          pltpu.VMEM((CH, d), _f32),
          pltpu.SemaphoreType.DMA,
      ],
  )
  def k(tab_hbm, idx_hbm, out_hbm, idx_v, rows_v, sem):
    c = lax.axis_index("c")
    s = lax.axis_index("s")
    w = s * NC + c
    nw = NC * NS

    def body(i, _):
      ch = i * nw + w
      pltpu.sync_copy(idx_hbm.at[ch], idx_v)
      pltpu.async_copy(tab_hbm.at[idx_v], rows_v, sem).wait()
      pltpu.sync_copy(rows_v, out_hbm.at[pl.ds(ch * CH, CH)])
      return 0

    lax.fori_loop(0, n_chunks // nw, body, 0)

  return k


_BLK = 256


def _tc_invdeg(nh_rows):
  blk = 1024

  def body(d0e_ref, d1e_ref, d0o_ref, d1o_ref, oe_ref, oo_ref):
    oe_ref[...] = 1.0 / jnp.maximum(d0e_ref[...] + d1e_ref[...], 1.0)
    oo_ref[...] = 1.0 / jnp.maximum(d0o_ref[...] + d1o_ref[...], 1.0)

  sp = pl.BlockSpec((blk,), lambda i: (i,))
  return pl.pallas_call(
      body,
      grid=(nh_rows // blk,),
      in_specs=[sp, sp, sp, sp],
      out_specs=[sp, sp],
      out_shape=[jax.ShapeDtypeStruct((nh_rows,), _f32),
                 jax.ShapeDtypeStruct((nh_rows,), _f32)],
  )


def _tc_frag_in(nh_rows, in_dim):
  h = 2 * HALF

  def body(x_ref, w_ref, b_ref, o_ref):
    w = w_ref[...]
    bb = b_ref[...][None, :]
    x = x_ref[...]
    o_ref[:, :h] = jnp.dot(x[:, :in_dim], w,
                           preferred_element_type=_f32) + bb
    o_ref[:, h:] = jnp.dot(x[:, in_dim:], w,
                           preferred_element_type=_f32) + bb

  return pl.pallas_call(
      body,
      grid=(nh_rows // _BLK,),
      in_specs=[
          pl.BlockSpec((_BLK, 2 * in_dim), lambda i: (i, 0)),
          pl.BlockSpec((in_dim, h), lambda i: (0, 0)),
          pl.BlockSpec((h,), lambda i: (0,)),
      ],
      out_specs=pl.BlockSpec((_BLK, 2 * h), lambda i: (i, 0)),
      out_shape=jax.ShapeDtypeStruct((nh_rows, 2 * h), _f32),
  )


def _layer_norm_rows(y, g, b):
  m = jnp.mean(y, axis=-1, keepdims=True)
  v = jnp.mean((y - m) * (y - m), axis=-1, keepdims=True)
  return (y - m) / jnp.sqrt(v + 1e-5) * g[None, :] + b[None, :]


def _tc_update(nh_rows, with_ctx):
  h = 2 * HALF

  def body(*refs):
    if with_ctx:
      (h_ref, a_ref, ie_ref, io_ref, ctx_ref, w_ref, b_ref,
       g_ref, bl_ref, o_ref) = refs
    else:
      (h_ref, a_ref, ie_ref, io_ref, w_ref, b_ref, g_ref, bl_ref,
       o_ref) = refs
    w = w_ref[...]
    bb = b_ref[...][None, :]
    hp = h_ref[...]
    ap = a_ref[...]

    def half_update(xh, ah, inv, ctxh):
      x = xh + ah * inv[:, None]
      y = (jnp.dot(x[:, :HALF], w[:HALF, :], preferred_element_type=_f32)
           + jnp.dot(x[:, HALF:], w[HALF:, :], preferred_element_type=_f32)
           + bb)
      if with_ctx:
        y = y + ctxh
      y = _layer_norm_rows(y, g_ref[...], bl_ref[...])
      return jnp.maximum(y, 0.0)

    ctx_lo = ctx_ref[...][:, :h] if with_ctx else None
    ctx_hi = ctx_ref[...][:, h:] if with_ctx else None
    o_ref[:, :h] = half_update(hp[:, :h], ap[:, :h], ie_ref[...], ctx_lo)
    o_ref[:, h:] = half_update(hp[:, h:], ap[:, h:], io_ref[...], ctx_hi)

  in_specs = [
      pl.BlockSpec((_BLK, 2 * h), lambda i: (i, 0)),
      pl.BlockSpec((_BLK, 2 * h), lambda i: (i, 0)),
      pl.BlockSpec((_BLK,), lambda i: (i,)),
      pl.BlockSpec((_BLK,), lambda i: (i,)),
  ]
  if with_ctx:
    in_specs.append(pl.BlockSpec((_BLK, 2 * h), lambda i: (i, 0)))
  in_specs += [
      pl.BlockSpec((h, h), lambda i: (0, 0)),
      pl.BlockSpec((h,), lambda i: (0,)),
      pl.BlockSpec((h,), lambda i: (0,)),
      pl.BlockSpec((h,), lambda i: (0,)),
  ]
  return pl.pallas_call(
      body,
      grid=(nh_rows // _BLK,),
      in_specs=in_specs,
      out_specs=pl.BlockSpec((_BLK, 2 * h), lambda i: (i, 0)),
      out_shape=jax.ShapeDtypeStruct((nh_rows, 2 * h), _f32),
  )


def _tc_pool_cond(pool_rows, g_count, td):
  h = 2 * HALF

  def body(p0_ref, p1_ref, c0_ref, c1_ref, t_ref, fw_ref, fb_ref,
           tw1_ref, tb1_ref, tw2_ref, tb2_ref, cw1_ref, cb1_ref,
           cw2_ref, cb2_ref, o_ref):
    pool = jnp.concatenate([p0_ref[0], p1_ref[0]], axis=1)
    cnt = (c0_ref[...] + c1_ref[...])[:, None]
    mean = pool / jnp.maximum(cnt, 1.0)
    fo = jnp.dot(mean, fw_ref[...],
                 preferred_element_type=_f32) + fb_ref[...][None, :]
    left = fo[:g_count, :]
    right = fo[g_count:2 * g_count, :]
    half = td // 2
    i = lax.broadcasted_iota(jnp.int32, (g_count, half), 1).astype(_f32)
    freqs = jnp.exp((-math.log(10000.0) / half) * i)
    a = t_ref[...][:, None] * freqs
    te = jnp.concatenate([jnp.sin(a), jnp.cos(a)], axis=1)
    th = jnp.dot(te, tw1_ref[...],
                 preferred_element_type=_f32) + tb1_ref[...][None, :]
    th = th * jax.nn.sigmoid(th)
    th = jnp.dot(th, tw2_ref[...],
                 preferred_element_type=_f32) + tb2_ref[...][None, :]
    ci = jnp.concatenate([left, right, th], axis=1)
    gc = jnp.dot(ci, cw1_ref[...],
                 preferred_element_type=_f32) + cb1_ref[...][None, :]
    gc = gc * jax.nn.sigmoid(gc)
    gc = jnp.dot(gc, cw2_ref[...],
                 preferred_element_type=_f32) + cb2_ref[...][None, :]
    o_ref[...] = gc

  full = lambda *shape: pl.BlockSpec(shape, lambda: tuple(0 for _ in shape))
  return pl.pallas_call(
      body,
      in_specs=[
          full(1, pool_rows, HALF), full(1, pool_rows, HALF),
          full(pool_rows), full(pool_rows),
          full(g_count),
          full(h, h), full(h),
          full(td, h), full(h), full(h, h), full(h),
          full(3 * h, h), full(h), full(h, h), full(h),
      ],
      out_specs=full(g_count, h),
      out_shape=jax.ShapeDtypeStruct((g_count, h), _f32),
  )


def _tc_linker_in(nh_rows, in_dim):
  h = 2 * HALF

  def body(x_ref, nte_ref, nto_ref, ctx_ref, w_ref, b_ref, o_ref):
    w = w_ref[...]
    bb = b_ref[...][None, :]
    x = x_ref[...]
    ctx = ctx_ref[...]

    def half_in(xh, nt, ctxh):
      y = jnp.dot(xh, w[:in_dim, :], preferred_element_type=_f32) + bb
      ntc = jnp.clip(nt, 0, 2)
      for kcls in range(3):
        y = y + (ntc == kcls).astype(_f32)[:, None] * w[in_dim + kcls][None]
      y = y + (nt > 0).astype(_f32)[:, None] * w[in_dim + 3][None]
      return y + ctxh

    o_ref[:, :h] = half_in(x[:, :in_dim], nte_ref[...], ctx[:, :h])
    o_ref[:, h:] = half_in(x[:, in_dim:], nto_ref[...], ctx[:, h:])

  return pl.pallas_call(
      body,
      grid=(nh_rows // _BLK,),
      in_specs=[
          pl.BlockSpec((_BLK, 2 * in_dim), lambda i: (i, 0)),
          pl.BlockSpec((_BLK,), lambda i: (i,)),
          pl.BlockSpec((_BLK,), lambda i: (i,)),
          pl.BlockSpec((_BLK, 2 * h), lambda i: (i, 0)),
          pl.BlockSpec((in_dim + 4, h), lambda i: (0, 0)),
          pl.BlockSpec((h,), lambda i: (0,)),
      ],
      out_specs=pl.BlockSpec((_BLK, 2 * h), lambda i: (i, 0)),
      out_shape=jax.ShapeDtypeStruct((nh_rows, 2 * h), _f32),
  )


def _tc_out(nh_rows, out_dim):
  h = 2 * HALF

  def body(h_ref, w_ref, b_ref, o_ref):
    w = w_ref[...]
    bb = b_ref[...][None, :]
    hp = h_ref[...]

    def half_out(xh):
      return (jnp.dot(xh[:, :HALF], w[:HALF, :],
                      preferred_element_type=_f32)
              + jnp.dot(xh[:, HALF:], w[HALF:, :],
                        preferred_element_type=_f32) + bb)

    o_ref[:, :out_dim] = half_out(hp[:, :h])
    o_ref[:, out_dim:] = half_out(hp[:, h:])

  return pl.pallas_call(
      body,
      grid=(nh_rows // _BLK,),
      in_specs=[
          pl.BlockSpec((_BLK, 2 * h), lambda i: (i, 0)),
          pl.BlockSpec((h, out_dim), lambda i: (0, 0)),
          pl.BlockSpec((out_dim,), lambda i: (0,)),
      ],
      out_specs=pl.BlockSpec((_BLK, 2 * out_dim), lambda i: (i, 0)),
      out_shape=jax.ShapeDtypeStruct((nh_rows, 2 * out_dim), _f32),
  )


def _pad_nodes_2d(a, np_rows, fill=0.0):
  return jnp.pad(a, ((0, np_rows - a.shape[0]), (0, 0)),
                 constant_values=fill)


def _pad_ids(ids, np_rows, fill):
  return jnp.pad(ids.astype(jnp.int32), (0, np_rows - ids.shape[0]),
                 constant_values=fill)


def _prep_edges(src, dst, ep, dummy_dst):
  e = src.shape[0]
  src = jnp.pad(src.astype(jnp.int32), (0, ep - e), constant_values=0)
  dst = jnp.pad(dst.astype(jnp.int32), (0, ep - e),
                constant_values=dummy_dst)
  return src.reshape(ep // CH, CH), dst.reshape(ep // CH, CH)


def kernel(x, t, linker_batch, linker_graph_ptr, linker_node_type,
           linker_edge_index, left_x, left_edge_index, left_batch,
           right_x, right_edge_index, right_batch, params):
  G = int(linker_graph_ptr.shape[0]) - 1
  N = x.shape[1]
  IN = x.shape[2]
  NF = left_x.shape[0]
  E = linker_edge_index.shape[1]
  EF = left_edge_index.shape[1]
  TD = params['time_W1'].shape[0]

  Np = _ceil_to(max(N, 2 * NF), NC * NS * CH)
  e_unit = NS * CH * 2 * MACRO
  Ep = _ceil_to(E, e_unit)
  EFp = _ceil_to(2 * EF, e_unit)
  PoolR = _ceil_to(2 * G + 1, CH)

  p = params
  fp = p['frag']
  Nh = Np // 2

  xL = _pad_nodes_2d(x[0], Np).reshape(Nh, 2 * IN)
  ntL = _pad_ids(linker_node_type, Np, 0)
  ntLe, ntLo = ntL[0::2], ntL[1::2]
  batL = _pad_ids(linker_batch, Np, 0).reshape(Np // CH, CH)
  srcL, dstL = _prep_edges(linker_edge_index[0], linker_edge_index[1],
                           Ep, Np)

  xF = _pad_nodes_2d(jnp.concatenate([left_x, right_x], axis=0),
                     Np).reshape(Nh, 2 * IN)
  srcF = jnp.concatenate([left_edge_index[0],
                          right_edge_index[0] + NF], axis=0)
  dstF = jnp.concatenate([left_edge_index[1],
                          right_edge_index[1] + NF], axis=0)
  srcF, dstF = _prep_edges(srcF, dstF, EFp, Np)
  bF = jnp.concatenate([left_batch, right_batch + G], axis=0)
  bF = _pad_ids(bF, Np, 2 * G).reshape(Np // CH, CH)

  degL = _sc_hist(Np, Ep // CH)(dstL)
  degF = _sc_hist(Np, EFp // CH)(dstF)
  cntF = _sc_hist(PoolR, Np // CH)(bF)
  invLe, invLo = _tc_invdeg(Nh)(degL[0, 0::2], degL[1, 0::2],
                                degL[0, 1::2], degL[1, 1::2])
  invFe, invFo = _tc_invdeg(Nh)(degF[0, 0::2], degF[1, 0::2],
                                degF[0, 1::2], degF[1, 1::2])

  agg_f = _sc_agg(Np, EFp // CH)
  upd_f = _tc_update(Nh, with_ctx=False)
  hp = _tc_frag_in(Nh, IN)(xF, fp['in_W'], fp['in_b'])
  for i in range(len(fp['conv_W'])):
    agg = agg_f(hp.reshape(2 * Np, HALF), srcF, dstF)
    hp = upd_f(hp, agg.reshape(Nh, 4 * HALF), invFe, invFo,
               fp['conv_W'][i], fp['conv_b'][i],
               fp['ln_g'][i], fp['ln_b'][i])

  pooled = _sc_pool(Np, PoolR)(hp.reshape(Np, NC, HALF), bF)
  graph_ctx = _tc_pool_cond(PoolR, G, TD)(
      pooled[0:1], pooled[1:2], cntF[0], cntF[1], t,
      fp['out_W'], fp['out_b'],
      p['time_W1'], p['time_b1'], p['time_W2'], p['time_b2'],
      p['cond_W1'], p['cond_b1'], p['cond_W2'], p['cond_b2'])

  ctx = _sc_gather_rows(Np, 2 * HALF)(graph_ctx, batL)
  ctxp = ctx.reshape(Nh, 4 * HALF)
  hp = _tc_linker_in(Nh, IN)(xL, ntLe, ntLo, ctxp, p['in_W'], p['in_b'])
  agg_l = _sc_agg(Np, Ep // CH)
  upd_l = _tc_update(Nh, with_ctx=True)
  for i in range(len(p['conv_W'])):
    agg = agg_l(hp.reshape(2 * Np, HALF), srcL, dstL)
    hp = upd_l(hp, agg.reshape(Nh, 4 * HALF), invLe, invLo, ctxp,
               p['conv_W'][i], p['conv_b'][i],
               p['ln_g'][i], p['ln_b'][i])

  out = _tc_out(Nh, p['out_W'].shape[1])(hp, p['out_W'], p['out_b'])
  OUTD = p['out_W'].shape[1]
  return out.reshape(Np, OUTD)[:N][None]


# speedup vs baseline: 1.8543x; 1.0837x over previous
import functools
import math

import jax
import jax.numpy as jnp
from jax import lax
from jax.experimental import pallas as pl
from jax.experimental.pallas import tpu as pltpu
from jax.experimental.pallas import tpu_sc as plsc

NC = 2
NS = 16
LANES = 16
CH = 128
MACRO = 4
HALF = 32

_f32 = jnp.float32


def _ceil_to(x, m):
  return (x + m - 1) // m * m


def _zero_fill(buf_ref, nrows, ncols):
  per_row = ncols // LANES

  def body(i, _):
    r = i // per_row
    c = (i % per_row) * LANES
    buf_ref[r, pl.ds(c, LANES)] = jnp.zeros((LANES,), _f32)
    return 0

  lax.fori_loop(0, nrows * per_row, body, 0)


def _const_fill(buf_ref, nrows, ncols, val):
  per_row = ncols // LANES

  def body(i, _):
    r = i // per_row
    c = (i % per_row) * LANES
    buf_ref[r, pl.ds(c, LANES)] = jnp.full((LANES,), val, _f32)
    return 0

  lax.fori_loop(0, nrows * per_row, body, 0)


def _sc_hist(np_rows, n_chunks):
  mesh = plsc.VectorSubcoreMesh(core_axis_name="c", subcore_axis_name="s",
                                num_cores=NC, num_subcores=NS)
  acc_rows = np_rows + CH
  out_chunks = np_rows // CH

  @functools.partial(
      pl.kernel,
      mesh=mesh,
      compiler_params=pltpu.CompilerParams(use_tc_tiling_on_sc=False),
      out_type=jax.ShapeDtypeStruct((NC, np_rows), _f32),
      scratch_types=[
          pltpu.VMEM_SHARED((acc_rows,), _f32),
          pltpu.VMEM((CH,), jnp.int32),
          pltpu.VMEM((CH,), _f32),
          pltpu.VMEM((CH,), _f32),
      ],
  )
  def k(dst_hbm, out_hbm, acc, idx_v, ones_v, buf_v):
    c = lax.axis_index("c")
    s = lax.axis_index("s")

    def fill(i, _):
      buf_v[pl.ds(i * LANES, LANES)] = jnp.zeros((LANES,), _f32)
      ones_v[pl.ds(i * LANES, LANES)] = jnp.ones((LANES,), _f32)
      return 0

    lax.fori_loop(0, CH // LANES, fill, 0)

    nz = acc_rows // CH

    def zbody(i, _):
      pltpu.sync_copy(buf_v, acc.at[pl.ds((i * NS + s) * CH, CH)])
      return 0

    lax.fori_loop(0, nz // NS, zbody, 0)

    @pl.when(s < nz % NS)
    def _():
      pltpu.sync_copy(buf_v, acc.at[pl.ds(((nz // NS) * NS + s) * CH, CH)])

    plsc.subcore_barrier()

    half = n_chunks // NC

    def ebody(i, _):
      ch = c * half + i * NS + s
      pltpu.sync_copy(dst_hbm.at[ch], idx_v)
      pltpu.sync_copy(ones_v, acc.at[idx_v], add=True)
      return 0

    lax.fori_loop(0, half // NS, ebody, 0)

    @pl.when(s < half % NS)
    def _():
      ch = c * half + (half // NS) * NS + s
      pltpu.sync_copy(dst_hbm.at[ch], idx_v)
      pltpu.sync_copy(ones_v, acc.at[idx_v], add=True)

    plsc.subcore_barrier()

    def obody(i, _):
      r = (i * NS + s) * CH
      pltpu.sync_copy(acc.at[pl.ds(r, CH)], buf_v)
      pltpu.sync_copy(buf_v, out_hbm.at[c, pl.ds(r, CH)])
      return 0

    lax.fori_loop(0, out_chunks // NS, obody, 0)

    @pl.when(s < out_chunks % NS)
    def _():
      r = ((out_chunks // NS) * NS + s) * CH
      pltpu.sync_copy(acc.at[pl.ds(r, CH)], buf_v)
      pltpu.sync_copy(buf_v, out_hbm.at[c, pl.ds(r, CH)])

  return k


def _sc_agg(np_rows, n_chunks):
  mesh = plsc.VectorSubcoreMesh(core_axis_name="c", subcore_axis_name="s",
                                num_cores=NC, num_subcores=NS)
  acc_rows = np_rows + CH
  out_chunks = np_rows // CH
  n_pair = n_chunks // (2 * MACRO)

  @functools.partial(
      pl.kernel,
      mesh=mesh,
      compiler_params=pltpu.CompilerParams(use_tc_tiling_on_sc=False),
      out_type=jax.ShapeDtypeStruct((np_rows, NC, HALF), _f32),
      scratch_types=[
          pltpu.VMEM_SHARED((acc_rows, HALF), _f32),
          pltpu.VMEM((MACRO, CH), jnp.int32),
          pltpu.VMEM((MACRO, CH), jnp.int32),
          pltpu.VMEM((MACRO, CH), jnp.int32),
          pltpu.VMEM((MACRO, CH), jnp.int32),
          pltpu.VMEM((MACRO * CH, HALF), _f32),
          pltpu.SemaphoreType.DMA,
          pltpu.SemaphoreType.DMA,
      ],
  )
  def k(h_hbm, src_hbm, dst_hbm, out_hbm, acc, src_a, dst_a, src_b, dst_b,
        rows_v, gsem, ssem):
    c = lax.axis_index("c")
    s = lax.axis_index("s")
    buf_v = rows_v.at[pl.ds(0, CH)]
    _zero_fill(rows_v, CH, HALF)

    nz = acc_rows // CH

    def zbody(i, _):
      pltpu.sync_copy(buf_v, acc.at[pl.ds((i * NS + s) * CH, CH)])
      return 0

    lax.fori_loop(0, nz // NS, zbody, 0)

    @pl.when(s < nz % NS)
    def _():
      pltpu.sync_copy(buf_v, acc.at[pl.ds(((nz // NS) * NS + s) * CH, CH)])

    plsc.subcore_barrier()

    n_tile_macros = n_chunks // MACRO // NS
    max_m = (n_chunks // MACRO - 1) * MACRO

    def run_macro(src_i):
      ga = [pltpu.async_copy(h_hbm.at[src_i.at[j]],
                             rows_v.at[pl.ds(j * CH, CH)], gsem)
            for j in range(MACRO)]
      return ga

    def load_idx(k, src_i, dst_i):
      m = jnp.minimum((k * NS + s) * MACRO, max_m)
      pltpu.sync_copy(src_hbm.at[pl.ds(m, MACRO)], src_i)
      pltpu.sync_copy(dst_hbm.at[pl.ds(m, MACRO)], dst_i)
      for j in range(MACRO):
        for t in range(CH // LANES):
          v = src_i[j, pl.ds(t * LANES, LANES)]
          src_i[j, pl.ds(t * LANES, LANES)] = v * 2 + c

    def drain_scatter(dst_i):
      sa = [pltpu.async_copy(rows_v.at[pl.ds(j * CH, CH)],
                             acc.at[dst_i.at[j]], ssem, add=True)
            for j in range(MACRO)]
      for d in sa:
        d.wait()

    load_idx(0, src_a, dst_a)

    def ebody(i, _):
      ga = run_macro(src_a)
      load_idx(2 * i + 1, src_b, dst_b)
      for d in ga:
        d.wait()
      drain_scatter(dst_a)
      gb = run_macro(src_b)
      load_idx(2 * i + 2, src_a, dst_a)
      for d in gb:
        d.wait()
      drain_scatter(dst_b)
      return 0

    lax.fori_loop(0, n_tile_macros // 2, ebody, 0)

    plsc.subcore_barrier()

    def obody(i, _):
      r = (i * NS + s) * CH
      pltpu.sync_copy(acc.at[pl.ds(r, CH)], buf_v)
      pltpu.sync_copy(buf_v, out_hbm.at[pl.ds(r, CH), c])
      return 0

    lax.fori_loop(0, out_chunks // NS, obody, 0)

    @pl.when(s < out_chunks % NS)
    def _():
      r = ((out_chunks // NS) * NS + s) * CH
      pltpu.sync_copy(acc.at[pl.ds(r, CH)], buf_v)
      pltpu.sync_copy(buf_v, out_hbm.at[pl.ds(r, CH), c])

  return k


def _sc_pool(np_rows, pool_rows):
  mesh = plsc.VectorSubcoreMesh(core_axis_name="c", subcore_axis_name="s",
                                num_cores=NC, num_subcores=NS)
  n_chunks = np_rows // CH
  out_chunks = pool_rows // CH

  @functools.partial(
      pl.kernel,
      mesh=mesh,
      compiler_params=pltpu.CompilerParams(use_tc_tiling_on_sc=False),
      out_type=jax.ShapeDtypeStruct((NC, pool_rows, HALF), _f32),
      scratch_types=[
          pltpu.VMEM_SHARED((pool_rows, HALF), _f32),
          pltpu.VMEM((CH,), jnp.int32),
          pltpu.VMEM((CH, HALF), _f32),
          pltpu.VMEM((CH, HALF), _f32),
      ],
  )
  def k(h_hbm, b_hbm, out_hbm, acc, idx_v, vals_v, buf_v):
    c = lax.axis_index("c")
    s = lax.axis_index("s")
    _zero_fill(buf_v, CH, HALF)

    @pl.when(s < out_chunks)
    def _():
      pltpu.sync_copy(buf_v, acc.at[pl.ds(s * CH, CH)])

    plsc.subcore_barrier()

    def ebody(i, _):
      ch = i * NS + s
      pltpu.sync_copy(b_hbm.at[ch], idx_v)
      pltpu.sync_copy(h_hbm.at[pl.ds(ch * CH, CH), c], vals_v)
      pltpu.sync_copy(vals_v, acc.at[idx_v], add=True)
      return 0

    lax.fori_loop(0, n_chunks // NS, ebody, 0)

    @pl.when(s < n_chunks % NS)
    def _():
      ch = (n_chunks // NS) * NS + s
      pltpu.sync_copy(b_hbm.at[ch], idx_v)
      pltpu.sync_copy(h_hbm.at[pl.ds(ch * CH, CH), c], vals_v)
      pltpu.sync_copy(vals_v, acc.at[idx_v], add=True)

    plsc.subcore_barrier()

    @pl.when(s < out_chunks)
    def _():
      pltpu.sync_copy(acc.at[pl.ds(s * CH, CH)], buf_v)
      pltpu.sync_copy(buf_v, out_hbm.at[c, pl.ds(s * CH, CH)])

  return k


def _sc_gather_rows(np_rows, d):
  mesh = plsc.VectorSubcoreMesh(core_axis_name="c", subcore_axis_name="s",
                                num_cores=NC, num_subcores=NS)
  n_chunks = np_rows // CH

  @functools.partial(
      pl.kernel,
      mesh=mesh,
      compiler_params=pltpu.CompilerParams(use_tc_tiling_on_sc=False),
      out_type=jax.ShapeDtypeStruct((np_rows, d), _f32),
      scratch_types=[
          pltpu.VMEM((CH,), jnp.int32),
          pltpu.VMEM((CH, d), _f32),
          pltpu.SemaphoreType.DMA,
      ],
  )
  def k(tab_hbm, idx_hbm, out_hbm, idx_v, rows_v, sem):
    c = lax.axis_index("c")
    s = lax.axis_index("s")
    w = s * NC + c
    nw = NC * NS

    def body(i, _):
      ch = i * nw + w
      pltpu.sync_copy(idx_hbm.at[ch], idx_v)
      pltpu.async_copy(tab_hbm.at[idx_v], rows_v, sem).wait()
      pltpu.sync_copy(rows_v, out_hbm.at[pl.ds(ch * CH, CH)])
      return 0

    lax.fori_loop(0, n_chunks // nw, body, 0)

  return k


_BLK = 512


def _tc_invdeg(nh_rows):
  blk = 1024

  def body(d0e_ref, d1e_ref, d0o_ref, d1o_ref, oe_ref, oo_ref):
    oe_ref[...] = 1.0 / jnp.maximum(d0e_ref[...] + d1e_ref[...], 1.0)
    oo_ref[...] = 1.0 / jnp.maximum(d0o_ref[...] + d1o_ref[...], 1.0)

  sp = pl.BlockSpec((blk,), lambda i: (i,))
  return pl.pallas_call(
      body,
      grid=(nh_rows // blk,),
      in_specs=[sp, sp, sp, sp],
      out_specs=[sp, sp],
      out_shape=[jax.ShapeDtypeStruct((nh_rows,), _f32),
                 jax.ShapeDtypeStruct((nh_rows,), _f32)],
  )


def _tc_frag_in(nh_rows, in_dim):
  h = 2 * HALF

  def body(x_ref, w_ref, b_ref, o_ref):
    w = w_ref[...]
    bb = b_ref[...][None, :]
    x = x_ref[...]
    o_ref[:, :h] = jnp.dot(x[:, :in_dim], w,
                           preferred_element_type=_f32) + bb
    o_ref[:, h:] = jnp.dot(x[:, in_dim:], w,
                           preferred_element_type=_f32) + bb

  return pl.pallas_call(
      body,
      grid=(nh_rows // _BLK,),
      in_specs=[
          pl.BlockSpec((_BLK, 2 * in_dim), lambda i: (i, 0)),
          pl.BlockSpec((in_dim, h), lambda i: (0, 0)),
          pl.BlockSpec((h,), lambda i: (0,)),
      ],
      out_specs=pl.BlockSpec((_BLK, 2 * h), lambda i: (i, 0)),
      out_shape=jax.ShapeDtypeStruct((nh_rows, 2 * h), _f32),
  )


def _layer_norm_rows(y, g, b):
  m = jnp.mean(y, axis=-1, keepdims=True)
  v = jnp.mean((y - m) * (y - m), axis=-1, keepdims=True)
  return (y - m) / jnp.sqrt(v + 1e-5) * g[None, :] + b[None, :]


def _tc_update(nh_rows, with_ctx):
  h = 2 * HALF

  def body(*refs):
    if with_ctx:
      (h_ref, a_ref, ie_ref, io_ref, ctx_ref, w_ref, b_ref,
       g_ref, bl_ref, o_ref) = refs
    else:
      (h_ref, a_ref, ie_ref, io_ref, w_ref, b_ref, g_ref, bl_ref,
       o_ref) = refs
    w = w_ref[...]
    bb = b_ref[...][None, :]
    hp = h_ref[...]
    ap = a_ref[...]

    def half_update(xh, ah, inv, ctxh):
      x = xh + ah * inv[:, None]
      y = (jnp.dot(x[:, :HALF], w[:HALF, :], preferred_element_type=_f32)
           + jnp.dot(x[:, HALF:], w[HALF:, :], preferred_element_type=_f32)
           + bb)
      if with_ctx:
        y = y + ctxh
      y = _layer_norm_rows(y, g_ref[...], bl_ref[...])
      return jnp.maximum(y, 0.0)

    ctx_lo = ctx_ref[...][:, :h] if with_ctx else None
    ctx_hi = ctx_ref[...][:, h:] if with_ctx else None
    o_ref[:, :h] = half_update(hp[:, :h], ap[:, :h], ie_ref[...], ctx_lo)
    o_ref[:, h:] = half_update(hp[:, h:], ap[:, h:], io_ref[...], ctx_hi)

  in_specs = [
      pl.BlockSpec((_BLK, 2 * h), lambda i: (i, 0)),
      pl.BlockSpec((_BLK, 2 * h), lambda i: (i, 0)),
      pl.BlockSpec((_BLK,), lambda i: (i,)),
      pl.BlockSpec((_BLK,), lambda i: (i,)),
  ]
  if with_ctx:
    in_specs.append(pl.BlockSpec((_BLK, 2 * h), lambda i: (i, 0)))
  in_specs += [
      pl.BlockSpec((h, h), lambda i: (0, 0)),
      pl.BlockSpec((h,), lambda i: (0,)),
      pl.BlockSpec((h,), lambda i: (0,)),
      pl.BlockSpec((h,), lambda i: (0,)),
  ]
  return pl.pallas_call(
      body,
      grid=(nh_rows // _BLK,),
      in_specs=in_specs,
      out_specs=pl.BlockSpec((_BLK, 2 * h), lambda i: (i, 0)),
      out_shape=jax.ShapeDtypeStruct((nh_rows, 2 * h), _f32),
  )


def _tc_pool_cond(pool_rows, g_count, td):
  h = 2 * HALF

  def body(p0_ref, p1_ref, c0_ref, c1_ref, t_ref, fw_ref, fb_ref,
           tw1_ref, tb1_ref, tw2_ref, tb2_ref, cw1_ref, cb1_ref,
           cw2_ref, cb2_ref, o_ref):
    pool = jnp.concatenate([p0_ref[0], p1_ref[0]], axis=1)
    cnt = (c0_ref[...] + c1_ref[...])[:, None]
    mean = pool / jnp.maximum(cnt, 1.0)
    fo = jnp.dot(mean, fw_ref[...],
                 preferred_element_type=_f32) + fb_ref[...][None, :]
    left = fo[:g_count, :]
    right = fo[g_count:2 * g_count, :]
    half = td // 2
    i = lax.broadcasted_iota(jnp.int32, (g_count, half), 1).astype(_f32)
    freqs = jnp.exp((-math.log(10000.0) / half) * i)
    a = t_ref[...][:, None] * freqs
    te = jnp.concatenate([jnp.sin(a), jnp.cos(a)], axis=1)
    th = jnp.dot(te, tw1_ref[...],
                 preferred_element_type=_f32) + tb1_ref[...][None, :]
    th = th * jax.nn.sigmoid(th)
    th = jnp.dot(th, tw2_ref[...],
                 preferred_element_type=_f32) + tb2_ref[...][None, :]
    ci = jnp.concatenate([left, right, th], axis=1)
    gc = jnp.dot(ci, cw1_ref[...],
                 preferred_element_type=_f32) + cb1_ref[...][None, :]
    gc = gc * jax.nn.sigmoid(gc)
    gc = jnp.dot(gc, cw2_ref[...],
                 preferred_element_type=_f32) + cb2_ref[...][None, :]
    o_ref[...] = gc

  full = lambda *shape: pl.BlockSpec(shape, lambda: tuple(0 for _ in shape))
  return pl.pallas_call(
      body,
      in_specs=[
          full(1, pool_rows, HALF), full(1, pool_rows, HALF),
          full(pool_rows), full(pool_rows),
          full(g_count),
          full(h, h), full(h),
          full(td, h), full(h), full(h, h), full(h),
          full(3 * h, h), full(h), full(h, h), full(h),
      ],
      out_specs=full(g_count, h),
      out_shape=jax.ShapeDtypeStruct((g_count, h), _f32),
  )


def _tc_linker_in(nh_rows, in_dim):
  h = 2 * HALF

  def body(x_ref, nte_ref, nto_ref, ctx_ref, w_ref, b_ref, o_ref):
    w = w_ref[...]
    bb = b_ref[...][None, :]
    x = x_ref[...]
    ctx = ctx_ref[...]

    def half_in(xh, nt, ctxh):
      y = jnp.dot(xh, w[:in_dim, :], preferred_element_type=_f32) + bb
      ntc = jnp.clip(nt, 0, 2)
      for kcls in range(3):
        y = y + (ntc == kcls).astype(_f32)[:, None] * w[in_dim + kcls][None]
      y = y + (nt > 0).astype(_f32)[:, None] * w[in_dim + 3][None]
      return y + ctxh

    o_ref[:, :h] = half_in(x[:, :in_dim], nte_ref[...], ctx[:, :h])
    o_ref[:, h:] = half_in(x[:, in_dim:], nto_ref[...], ctx[:, h:])

  return pl.pallas_call(
      body,
      grid=(nh_rows // _BLK,),
      in_specs=[
          pl.BlockSpec((_BLK, 2 * in_dim), lambda i: (i, 0)),
          pl.BlockSpec((_BLK,), lambda i: (i,)),
          pl.BlockSpec((_BLK,), lambda i: (i,)),
          pl.BlockSpec((_BLK, 2 * h), lambda i: (i, 0)),
          pl.BlockSpec((in_dim + 4, h), lambda i: (0, 0)),
          pl.BlockSpec((h,), lambda i: (0,)),
      ],
      out_specs=pl.BlockSpec((_BLK, 2 * h), lambda i: (i, 0)),
      out_shape=jax.ShapeDtypeStruct((nh_rows, 2 * h), _f32),
  )


def _tc_out(nh_rows, out_dim):
  h = 2 * HALF

  def body(h_ref, w_ref, b_ref, o_ref):
    w = w_ref[...]
    bb = b_ref[...][None, :]
    hp = h_ref[...]

    def half_out(xh):
      return (jnp.dot(xh[:, :HALF], w[:HALF, :],
                      preferred_element_type=_f32)
              + jnp.dot(xh[:, HALF:], w[HALF:, :],
                        preferred_element_type=_f32) + bb)

    o_ref[:, :out_dim] = half_out(hp[:, :h])
    o_ref[:, out_dim:] = half_out(hp[:, h:])

  return pl.pallas_call(
      body,
      grid=(nh_rows // _BLK,),
      in_specs=[
          pl.BlockSpec((_BLK, 2 * h), lambda i: (i, 0)),
          pl.BlockSpec((h, out_dim), lambda i: (0, 0)),
          pl.BlockSpec((out_dim,), lambda i: (0,)),
      ],
      out_specs=pl.BlockSpec((_BLK, 2 * out_dim), lambda i: (i, 0)),
      out_shape=jax.ShapeDtypeStruct((nh_rows, 2 * out_dim), _f32),
  )


def _pad_nodes_2d(a, np_rows, fill=0.0):
  return jnp.pad(a, ((0, np_rows - a.shape[0]), (0, 0)),
                 constant_values=fill)


def _pad_ids(ids, np_rows, fill):
  return jnp.pad(ids.astype(jnp.int32), (0, np_rows - ids.shape[0]),
                 constant_values=fill)


def _prep_edges(src, dst, ep, dummy_dst):
  e = src.shape[0]
  src = jnp.pad(src.astype(jnp.int32), (0, ep - e), constant_values=0)
  dst = jnp.pad(dst.astype(jnp.int32), (0, ep - e),
                constant_values=dummy_dst)
  return src.reshape(ep // CH, CH), dst.reshape(ep // CH, CH)


def kernel(x, t, linker_batch, linker_graph_ptr, linker_node_type,
           linker_edge_index, left_x, left_edge_index, left_batch,
           right_x, right_edge_index, right_batch, params):
  G = int(linker_graph_ptr.shape[0]) - 1
  N = x.shape[1]
  IN = x.shape[2]
  NF = left_x.shape[0]
  E = linker_edge_index.shape[1]
  EF = left_edge_index.shape[1]
  TD = params['time_W1'].shape[0]

  Np = _ceil_to(max(N, 2 * NF), NC * NS * CH)
  e_unit = NS * CH * 2 * MACRO
  Ep = _ceil_to(E, e_unit)
  EFp = _ceil_to(2 * EF, e_unit)
  PoolR = _ceil_to(2 * G + 1, CH)

  p = params
  fp = p['frag']
  Nh = Np // 2

  xL = _pad_nodes_2d(x[0], Np).reshape(Nh, 2 * IN)
  ntL = _pad_ids(linker_node_type, Np, 0)
  ntLe, ntLo = ntL[0::2], ntL[1::2]
  batL = _pad_ids(linker_batch, Np, 0).reshape(Np // CH, CH)
  srcL, dstL = _prep_edges(linker_edge_index[0], linker_edge_index[1],
                           Ep, Np)

  xF = _pad_nodes_2d(jnp.concatenate([left_x, right_x], axis=0),
                     Np).reshape(Nh, 2 * IN)
  srcF = jnp.concatenate([left_edge_index[0],
                          right_edge_index[0] + NF], axis=0)
  dstF = jnp.concatenate([left_edge_index[1],
                          right_edge_index[1] + NF], axis=0)
  srcF, dstF = _prep_edges(srcF, dstF, EFp, Np)
  bF = jnp.concatenate([left_batch, right_batch + G], axis=0)
  bF = _pad_ids(bF, Np, 2 * G).reshape(Np // CH, CH)

  degL = _sc_hist(Np, Ep // CH)(dstL)
  degF = _sc_hist(Np, EFp // CH)(dstF)
  cntF = _sc_hist(PoolR, Np // CH)(bF)
  invLe, invLo = _tc_invdeg(Nh)(degL[0, 0::2], degL[1, 0::2],
                                degL[0, 1::2], degL[1, 1::2])
  invFe, invFo = _tc_invdeg(Nh)(degF[0, 0::2], degF[1, 0::2],
                                degF[0, 1::2], degF[1, 1::2])

  agg_f = _sc_agg(Np, EFp // CH)
  upd_f = _tc_update(Nh, with_ctx=False)
  hp = _tc_frag_in(Nh, IN)(xF, fp['in_W'], fp['in_b'])
  for i in range(len(fp['conv_W'])):
    agg = agg_f(hp.reshape(2 * Np, HALF), srcF, dstF)
    hp = upd_f(hp, agg.reshape(Nh, 4 * HALF), invFe, invFo,
               fp['conv_W'][i], fp['conv_b'][i],
               fp['ln_g'][i], fp['ln_b'][i])

  pooled = _sc_pool(Np, PoolR)(hp.reshape(Np, NC, HALF), bF)
  graph_ctx = _tc_pool_cond(PoolR, G, TD)(
      pooled[0:1], pooled[1:2], cntF[0], cntF[1], t,
      fp['out_W'], fp['out_b'],
      p['time_W1'], p['time_b1'], p['time_W2'], p['time_b2'],
      p['cond_W1'], p['cond_b1'], p['cond_W2'], p['cond_b2'])

  ctx = _sc_gather_rows(Np, 2 * HALF)(graph_ctx, batL)
  ctxp = ctx.reshape(Nh, 4 * HALF)
  hp = _tc_linker_in(Nh, IN)(xL, ntLe, ntLo, ctxp, p['in_W'], p['in_b'])
  agg_l = _sc_agg(Np, Ep // CH)
  upd_l = _tc_update(Nh, with_ctx=True)
  for i in range(len(p['conv_W'])):
    agg = agg_l(hp.reshape(2 * Np, HALF), srcL, dstL)
    hp = upd_l(hp, agg.reshape(Nh, 4 * HALF), invLe, invLo, ctxp,
               p['conv_W'][i], p['conv_b'][i],
               p['ln_g'][i], p['ln_b'][i])

  out = _tc_out(Nh, p['out_W'].shape[1])(hp, p['out_W'], p['out_b'])
  OUTD = p['out_W'].shape[1]
  return out.reshape(Np, OUTD)[:N][None]


# speedup vs baseline: 1.9220x; 1.0365x over previous
import functools
import math

import jax
import jax.numpy as jnp
from jax import lax
from jax.experimental import pallas as pl
from jax.experimental.pallas import tpu as pltpu
from jax.experimental.pallas import tpu_sc as plsc

NC = 2
NS = 16
LANES = 16
CH = 128
MACRO = 4
HALF = 32

_f32 = jnp.float32


def _ceil_to(x, m):
  return (x + m - 1) // m * m


def _zero_fill(buf_ref, nrows, ncols):
  per_row = ncols // LANES

  def body(i, _):
    r = i // per_row
    c = (i % per_row) * LANES
    buf_ref[r, pl.ds(c, LANES)] = jnp.zeros((LANES,), _f32)
    return 0

  lax.fori_loop(0, nrows * per_row, body, 0)


def _const_fill(buf_ref, nrows, ncols, val):
  per_row = ncols // LANES

  def body(i, _):
    r = i // per_row
    c = (i % per_row) * LANES
    buf_ref[r, pl.ds(c, LANES)] = jnp.full((LANES,), val, _f32)
    return 0

  lax.fori_loop(0, nrows * per_row, body, 0)


def _sc_hist(np_rows, n_chunks):
  mesh = plsc.VectorSubcoreMesh(core_axis_name="c", subcore_axis_name="s",
                                num_cores=NC, num_subcores=NS)
  acc_rows = np_rows + CH
  out_chunks = np_rows // CH

  @functools.partial(
      pl.kernel,
      mesh=mesh,
      compiler_params=pltpu.CompilerParams(use_tc_tiling_on_sc=False),
      out_type=jax.ShapeDtypeStruct((NC, np_rows), _f32),
      scratch_types=[
          pltpu.VMEM_SHARED((acc_rows,), _f32),
          pltpu.VMEM((CH,), jnp.int32),
          pltpu.VMEM((CH,), _f32),
          pltpu.VMEM((CH,), _f32),
      ],
  )
  def k(dst_hbm, out_hbm, acc, idx_v, ones_v, buf_v):
    c = lax.axis_index("c")
    s = lax.axis_index("s")

    def fill(i, _):
      buf_v[pl.ds(i * LANES, LANES)] = jnp.zeros((LANES,), _f32)
      ones_v[pl.ds(i * LANES, LANES)] = jnp.ones((LANES,), _f32)
      return 0

    lax.fori_loop(0, CH // LANES, fill, 0)

    nz = acc_rows // CH

    def zbody(i, _):
      pltpu.sync_copy(buf_v, acc.at[pl.ds((i * NS + s) * CH, CH)])
      return 0

    lax.fori_loop(0, nz // NS, zbody, 0)

    @pl.when(s < nz % NS)
    def _():
      pltpu.sync_copy(buf_v, acc.at[pl.ds(((nz // NS) * NS + s) * CH, CH)])

    plsc.subcore_barrier()

    half = n_chunks // NC

    def ebody(i, _):
      ch = c * half + i * NS + s
      pltpu.sync_copy(dst_hbm.at[ch], idx_v)
      pltpu.sync_copy(ones_v, acc.at[idx_v], add=True)
      return 0

    lax.fori_loop(0, half // NS, ebody, 0)

    @pl.when(s < half % NS)
    def _():
      ch = c * half + (half // NS) * NS + s
      pltpu.sync_copy(dst_hbm.at[ch], idx_v)
      pltpu.sync_copy(ones_v, acc.at[idx_v], add=True)

    plsc.subcore_barrier()

    def obody(i, _):
      r = (i * NS + s) * CH
      pltpu.sync_copy(acc.at[pl.ds(r, CH)], buf_v)
      pltpu.sync_copy(buf_v, out_hbm.at[c, pl.ds(r, CH)])
      return 0

    lax.fori_loop(0, out_chunks // NS, obody, 0)

    @pl.when(s < out_chunks % NS)
    def _():
      r = ((out_chunks // NS) * NS + s) * CH
      pltpu.sync_copy(acc.at[pl.ds(r, CH)], buf_v)
      pltpu.sync_copy(buf_v, out_hbm.at[c, pl.ds(r, CH)])

  return k


def _sc_agg(np_rows, n_chunks):
  mesh = plsc.VectorSubcoreMesh(core_axis_name="c", subcore_axis_name="s",
                                num_cores=NC, num_subcores=NS)
  acc_rows = np_rows + CH
  out_chunks = np_rows // CH
  n_pair = n_chunks // (2 * MACRO)

  @functools.partial(
      pl.kernel,
      mesh=mesh,
      compiler_params=pltpu.CompilerParams(use_tc_tiling_on_sc=False),
      out_type=jax.ShapeDtypeStruct((np_rows, NC, HALF), _f32),
      scratch_types=[
          pltpu.VMEM_SHARED((acc_rows, HALF), _f32),
          pltpu.VMEM((MACRO, CH), jnp.int32),
          pltpu.VMEM((MACRO, CH), jnp.int32),
          pltpu.VMEM((MACRO, CH), jnp.int32),
          pltpu.VMEM((MACRO, CH), jnp.int32),
          pltpu.VMEM((MACRO * CH, HALF), _f32),
          pltpu.SemaphoreType.DMA,
          pltpu.SemaphoreType.DMA,
      ],
  )
  def k(h_hbm, src_hbm, dst_hbm, out_hbm, acc, src_a, dst_a, src_b, dst_b,
        rows_v, gsem, ssem):
    c = lax.axis_index("c")
    s = lax.axis_index("s")
    buf_v = rows_v.at[pl.ds(0, CH)]
    _zero_fill(rows_v, CH, HALF)

    nz = acc_rows // CH

    def zbody(i, _):
      pltpu.sync_copy(buf_v, acc.at[pl.ds((i * NS + s) * CH, CH)])
      return 0

    lax.fori_loop(0, nz // NS, zbody, 0)

    @pl.when(s < nz % NS)
    def _():
      pltpu.sync_copy(buf_v, acc.at[pl.ds(((nz // NS) * NS + s) * CH, CH)])

    plsc.subcore_barrier()

    n_tile_macros = n_chunks // MACRO // NS
    max_m = (n_chunks // MACRO - 1) * MACRO

    def run_macro(src_i):
      ga = [pltpu.async_copy(h_hbm.at[src_i.at[j]],
                             rows_v.at[pl.ds(j * CH, CH)], gsem)
            for j in range(MACRO)]
      return ga

    def load_idx(k, src_i, dst_i):
      m = jnp.minimum((k * NS + s) * MACRO, max_m)
      pltpu.sync_copy(src_hbm.at[pl.ds(m, MACRO)], src_i)
      pltpu.sync_copy(dst_hbm.at[pl.ds(m, MACRO)], dst_i)
      for j in range(MACRO):
        for t in range(CH // LANES):
          v = src_i[j, pl.ds(t * LANES, LANES)]
          src_i[j, pl.ds(t * LANES, LANES)] = v * 2 + c

    def drain_scatter(dst_i):
      sa = [pltpu.async_copy(rows_v.at[pl.ds(j * CH, CH)],
                             acc.at[dst_i.at[j]], ssem, add=True)
            for j in range(MACRO)]
      for d in sa:
        d.wait()

    load_idx(0, src_a, dst_a)

    def ebody(i, _):
      ga = run_macro(src_a)
      load_idx(2 * i + 1, src_b, dst_b)
      for d in ga:
        d.wait()
      drain_scatter(dst_a)
      gb = run_macro(src_b)
      load_idx(2 * i + 2, src_a, dst_a)
      for d in gb:
        d.wait()
      drain_scatter(dst_b)
      return 0

    lax.fori_loop(0, n_tile_macros // 2, ebody, 0)

    plsc.subcore_barrier()

    def obody(i, _):
      r = (i * NS + s) * CH
      pltpu.sync_copy(acc.at[pl.ds(r, CH)], buf_v)
      pltpu.sync_copy(buf_v, out_hbm.at[pl.ds(r, CH), c])
      return 0

    lax.fori_loop(0, out_chunks // NS, obody, 0)

    @pl.when(s < out_chunks % NS)
    def _():
      r = ((out_chunks // NS) * NS + s) * CH
      pltpu.sync_copy(acc.at[pl.ds(r, CH)], buf_v)
      pltpu.sync_copy(buf_v, out_hbm.at[pl.ds(r, CH), c])

  return k


def _sc_pool(np_rows, pool_rows):
  mesh = plsc.VectorSubcoreMesh(core_axis_name="c", subcore_axis_name="s",
                                num_cores=NC, num_subcores=NS)
  n_chunks = np_rows // CH
  out_chunks = pool_rows // CH

  @functools.partial(
      pl.kernel,
      mesh=mesh,
      compiler_params=pltpu.CompilerParams(use_tc_tiling_on_sc=False),
      out_type=jax.ShapeDtypeStruct((NC, pool_rows, HALF), _f32),
      scratch_types=[
          pltpu.VMEM_SHARED((pool_rows, HALF), _f32),
          pltpu.VMEM((CH,), jnp.int32),
          pltpu.VMEM((CH, HALF), _f32),
          pltpu.VMEM((CH, HALF), _f32),
      ],
  )
  def k(h_hbm, b_hbm, out_hbm, acc, idx_v, vals_v, buf_v):
    c = lax.axis_index("c")
    s = lax.axis_index("s")
    _zero_fill(buf_v, CH, HALF)

    @pl.when(s < out_chunks)
    def _():
      pltpu.sync_copy(buf_v, acc.at[pl.ds(s * CH, CH)])

    plsc.subcore_barrier()

    def ebody(i, _):
      ch = i * NS + s
      pltpu.sync_copy(b_hbm.at[ch], idx_v)
      pltpu.sync_copy(h_hbm.at[pl.ds(ch * CH, CH), c], vals_v)
      pltpu.sync_copy(vals_v, acc.at[idx_v], add=True)
      return 0

    lax.fori_loop(0, n_chunks // NS, ebody, 0)

    @pl.when(s < n_chunks % NS)
    def _():
      ch = (n_chunks // NS) * NS + s
      pltpu.sync_copy(b_hbm.at[ch], idx_v)
      pltpu.sync_copy(h_hbm.at[pl.ds(ch * CH, CH), c], vals_v)
      pltpu.sync_copy(vals_v, acc.at[idx_v], add=True)

    plsc.subcore_barrier()

    @pl.when(s < out_chunks)
    def _():
      pltpu.sync_copy(acc.at[pl.ds(s * CH, CH)], buf_v)
      pltpu.sync_copy(buf_v, out_hbm.at[c, pl.ds(s * CH, CH)])

  return k


def _sc_gather_rows(np_rows, d):
  mesh = plsc.VectorSubcoreMesh(core_axis_name="c", subcore_axis_name="s",
                                num_cores=NC, num_subcores=NS)
  n_chunks = np_rows // CH

  @functools.partial(
      pl.kernel,
      mesh=mesh,
      compiler_params=pltpu.CompilerParams(use_tc_tiling_on_sc=False),
      out_type=jax.ShapeDtypeStruct((np_rows, d), _f32),
      scratch_types=[
          pltpu.VMEM((CH,), jnp.int32),
          pltpu.VMEM((CH, d), _f32),
          pltpu.SemaphoreType.DMA,
      ],
  )
  def k(tab_hbm, idx_hbm, out_hbm, idx_v, rows_v, sem):
    c = lax.axis_index("c")
    s = lax.axis_index("s")
    w = s * NC + c
    nw = NC * NS

    def body(i, _):
      ch = i * nw + w
      pltpu.sync_copy(idx_hbm.at[ch], idx_v)
      pltpu.async_copy(tab_hbm.at[idx_v], rows_v, sem).wait()
      pltpu.sync_copy(rows_v, out_hbm.at[pl.ds(ch * CH, CH)])
      return 0

    lax.fori_loop(0, n_chunks // nw, body, 0)

  return k


_BLK = 1024


def _tc_invdeg(nh_rows):
  blk = 1024

  def body(d0e_ref, d1e_ref, d0o_ref, d1o_ref, oe_ref, oo_ref):
    oe_ref[...] = 1.0 / jnp.maximum(d0e_ref[...] + d1e_ref[...], 1.0)
    oo_ref[...] = 1.0 / jnp.maximum(d0o_ref[...] + d1o_ref[...], 1.0)

  sp = pl.BlockSpec((blk,), lambda i: (i,))
  return pl.pallas_call(
      body,
      grid=(nh_rows // blk,),
      in_specs=[sp, sp, sp, sp],
      out_specs=[sp, sp],
      out_shape=[jax.ShapeDtypeStruct((nh_rows,), _f32),
                 jax.ShapeDtypeStruct((nh_rows,), _f32)],
  )


def _tc_frag_in(nh_rows, in_dim):
  h = 2 * HALF

  def body(x_ref, w_ref, b_ref, o_ref):
    w = w_ref[...]
    bb = b_ref[...][None, :]
    x = x_ref[...]
    o_ref[:, :h] = jnp.dot(x[:, :in_dim], w,
                           preferred_element_type=_f32) + bb
    o_ref[:, h:] = jnp.dot(x[:, in_dim:], w,
                           preferred_element_type=_f32) + bb

  return pl.pallas_call(
      body,
      grid=(nh_rows // _BLK,),
      in_specs=[
          pl.BlockSpec((_BLK, 2 * in_dim), lambda i: (i, 0)),
          pl.BlockSpec((in_dim, h), lambda i: (0, 0)),
          pl.BlockSpec((h,), lambda i: (0,)),
      ],
      out_specs=pl.BlockSpec((_BLK, 2 * h), lambda i: (i, 0)),
      out_shape=jax.ShapeDtypeStruct((nh_rows, 2 * h), _f32),
  )


def _layer_norm_rows(y, g, b):
  m = jnp.mean(y, axis=-1, keepdims=True)
  v = jnp.mean((y - m) * (y - m), axis=-1, keepdims=True)
  return (y - m) / jnp.sqrt(v + 1e-5) * g[None, :] + b[None, :]


def _tc_update(nh_rows, with_ctx):
  h = 2 * HALF

  def body(*refs):
    if with_ctx:
      (h_ref, a_ref, ie_ref, io_ref, ctx_ref, w_ref, b_ref,
       g_ref, bl_ref, o_ref) = refs
    else:
      (h_ref, a_ref, ie_ref, io_ref, w_ref, b_ref, g_ref, bl_ref,
       o_ref) = refs
    w = w_ref[...]
    bb = b_ref[...][None, :]
    hp = h_ref[...]
    ap = a_ref[...]

    def half_update(xh, ah, inv, ctxh):
      x = xh + ah * inv[:, None]
      y = (jnp.dot(x[:, :HALF], w[:HALF, :], preferred_element_type=_f32)
           + jnp.dot(x[:, HALF:], w[HALF:, :], preferred_element_type=_f32)
           + bb)
      if with_ctx:
        y = y + ctxh
      y = _layer_norm_rows(y, g_ref[...], bl_ref[...])
      return jnp.maximum(y, 0.0)

    ctx_lo = ctx_ref[...][:, :h] if with_ctx else None
    ctx_hi = ctx_ref[...][:, h:] if with_ctx else None
    o_ref[:, :h] = half_update(hp[:, :h], ap[:, :h], ie_ref[...], ctx_lo)
    o_ref[:, h:] = half_update(hp[:, h:], ap[:, h:], io_ref[...], ctx_hi)

  in_specs = [
      pl.BlockSpec((_BLK, 2 * h), lambda i: (i, 0)),
      pl.BlockSpec((_BLK, 2 * h), lambda i: (i, 0)),
      pl.BlockSpec((_BLK,), lambda i: (i,)),
      pl.BlockSpec((_BLK,), lambda i: (i,)),
  ]
  if with_ctx:
    in_specs.append(pl.BlockSpec((_BLK, 2 * h), lambda i: (i, 0)))
  in_specs += [
      pl.BlockSpec((h, h), lambda i: (0, 0)),
      pl.BlockSpec((h,), lambda i: (0,)),
      pl.BlockSpec((h,), lambda i: (0,)),
      pl.BlockSpec((h,), lambda i: (0,)),
  ]
  return pl.pallas_call(
      body,
      grid=(nh_rows // _BLK,),
      in_specs=in_specs,
      out_specs=pl.BlockSpec((_BLK, 2 * h), lambda i: (i, 0)),
      out_shape=jax.ShapeDtypeStruct((nh_rows, 2 * h), _f32),
  )


def _tc_pool_cond(pool_rows, g_count, td):
  h = 2 * HALF

  def body(p0_ref, p1_ref, c0_ref, c1_ref, t_ref, fw_ref, fb_ref,
           tw1_ref, tb1_ref, tw2_ref, tb2_ref, cw1_ref, cb1_ref,
           cw2_ref, cb2_ref, o_ref):
    pool = jnp.concatenate([p0_ref[0], p1_ref[0]], axis=1)
    cnt = (c0_ref[...] + c1_ref[...])[:, None]
    mean = pool / jnp.maximum(cnt, 1.0)
    fo = jnp.dot(mean, fw_ref[...],
                 preferred_element_type=_f32) + fb_ref[...][None, :]
    left = fo[:g_count, :]
    right = fo[g_count:2 * g_count, :]
    half = td // 2
    i = lax.broadcasted_iota(jnp.int32, (g_count, half), 1).astype(_f32)
    freqs = jnp.exp((-math.log(10000.0) / half) * i)
    a = t_ref[...][:, None] * freqs
    te = jnp.concatenate([jnp.sin(a), jnp.cos(a)], axis=1)
    th = jnp.dot(te, tw1_ref[...],
                 preferred_element_type=_f32) + tb1_ref[...][None, :]
    th = th * jax.nn.sigmoid(th)
    th = jnp.dot(th, tw2_ref[...],
                 preferred_element_type=_f32) + tb2_ref[...][None, :]
    ci = jnp.concatenate([left, right, th], axis=1)
    gc = jnp.dot(ci, cw1_ref[...],
                 preferred_element_type=_f32) + cb1_ref[...][None, :]
    gc = gc * jax.nn.sigmoid(gc)
    gc = jnp.dot(gc, cw2_ref[...],
                 preferred_element_type=_f32) + cb2_ref[...][None, :]
    o_ref[...] = gc

  full = lambda *shape: pl.BlockSpec(shape, lambda: tuple(0 for _ in shape))
  return pl.pallas_call(
      body,
      in_specs=[
          full(1, pool_rows, HALF), full(1, pool_rows, HALF),
          full(pool_rows), full(pool_rows),
          full(g_count),
          full(h, h), full(h),
          full(td, h), full(h), full(h, h), full(h),
          full(3 * h, h), full(h), full(h, h), full(h),
      ],
      out_specs=full(g_count, h),
      out_shape=jax.ShapeDtypeStruct((g_count, h), _f32),
  )


def _tc_linker_in(nh_rows, in_dim):
  h = 2 * HALF

  def body(x_ref, nte_ref, nto_ref, ctx_ref, w_ref, b_ref, o_ref):
    w = w_ref[...]
    bb = b_ref[...][None, :]
    x = x_ref[...]
    ctx = ctx_ref[...]

    def half_in(xh, nt, ctxh):
      y = jnp.dot(xh, w[:in_dim, :], preferred_element_type=_f32) + bb
      ntc = jnp.clip(nt, 0, 2)
      for kcls in range(3):
        y = y + (ntc == kcls).astype(_f32)[:, None] * w[in_dim + kcls][None]
      y = y + (nt > 0).astype(_f32)[:, None] * w[in_dim + 3][None]
      return y + ctxh

    o_ref[:, :h] = half_in(x[:, :in_dim], nte_ref[...], ctx[:, :h])
    o_ref[:, h:] = half_in(x[:, in_dim:], nto_ref[...], ctx[:, h:])

  return pl.pallas_call(
      body,
      grid=(nh_rows // _BLK,),
      in_specs=[
          pl.BlockSpec((_BLK, 2 * in_dim), lambda i: (i, 0)),
          pl.BlockSpec((_BLK,), lambda i: (i,)),
          pl.BlockSpec((_BLK,), lambda i: (i,)),
          pl.BlockSpec((_BLK, 2 * h), lambda i: (i, 0)),
          pl.BlockSpec((in_dim + 4, h), lambda i: (0, 0)),
          pl.BlockSpec((h,), lambda i: (0,)),
      ],
      out_specs=pl.BlockSpec((_BLK, 2 * h), lambda i: (i, 0)),
      out_shape=jax.ShapeDtypeStruct((nh_rows, 2 * h), _f32),
  )


def _tc_out(nh_rows, out_dim):
  h = 2 * HALF

  def body(h_ref, w_ref, b_ref, o_ref):
    w = w_ref[...]
    bb = b_ref[...][None, :]
    hp = h_ref[...]

    def half_out(xh):
      return (jnp.dot(xh[:, :HALF], w[:HALF, :],
                      preferred_element_type=_f32)
              + jnp.dot(xh[:, HALF:], w[HALF:, :],
                        preferred_element_type=_f32) + bb)

    o_ref[:, :out_dim] = half_out(hp[:, :h])
    o_ref[:, out_dim:] = half_out(hp[:, h:])

  return pl.pallas_call(
      body,
      grid=(nh_rows // _BLK,),
      in_specs=[
          pl.BlockSpec((_BLK, 2 * h), lambda i: (i, 0)),
          pl.BlockSpec((h, out_dim), lambda i: (0, 0)),
          pl.BlockSpec((out_dim,), lambda i: (0,)),
      ],
      out_specs=pl.BlockSpec((_BLK, 2 * out_dim), lambda i: (i, 0)),
      out_shape=jax.ShapeDtypeStruct((nh_rows, 2 * out_dim), _f32),
  )


def _pad_nodes_2d(a, np_rows, fill=0.0):
  return jnp.pad(a, ((0, np_rows - a.shape[0]), (0, 0)),
                 constant_values=fill)


def _pad_ids(ids, np_rows, fill):
  return jnp.pad(ids.astype(jnp.int32), (0, np_rows - ids.shape[0]),
                 constant_values=fill)


def _prep_edges(src, dst, ep, dummy_dst):
  e = src.shape[0]
  src = jnp.pad(src.astype(jnp.int32), (0, ep - e), constant_values=0)
  dst = jnp.pad(dst.astype(jnp.int32), (0, ep - e),
                constant_values=dummy_dst)
  return src.reshape(ep // CH, CH), dst.reshape(ep // CH, CH)


def kernel(x, t, linker_batch, linker_graph_ptr, linker_node_type,
           linker_edge_index, left_x, left_edge_index, left_batch,
           right_x, right_edge_index, right_batch, params):
  G = int(linker_graph_ptr.shape[0]) - 1
  N = x.shape[1]
  IN = x.shape[2]
  NF = left_x.shape[0]
  E = linker_edge_index.shape[1]
  EF = left_edge_index.shape[1]
  TD = params['time_W1'].shape[0]

  Np = _ceil_to(max(N, 2 * NF), NC * NS * CH)
  e_unit = NS * CH * 2 * MACRO
  Ep = _ceil_to(E, e_unit)
  EFp = _ceil_to(2 * EF, e_unit)
  PoolR = _ceil_to(2 * G + 1, CH)

  p = params
  fp = p['frag']
  Nh = Np // 2

  xL = _pad_nodes_2d(x[0], Np).reshape(Nh, 2 * IN)
  ntL = _pad_ids(linker_node_type, Np, 0)
  ntLe, ntLo = ntL[0::2], ntL[1::2]
  batL = _pad_ids(linker_batch, Np, 0).reshape(Np // CH, CH)
  srcL, dstL = _prep_edges(linker_edge_index[0], linker_edge_index[1],
                           Ep, Np)

  xF = _pad_nodes_2d(jnp.concatenate([left_x, right_x], axis=0),
                     Np).reshape(Nh, 2 * IN)
  srcF = jnp.concatenate([left_edge_index[0],
                          right_edge_index[0] + NF], axis=0)
  dstF = jnp.concatenate([left_edge_index[1],
                          right_edge_index[1] + NF], axis=0)
  srcF, dstF = _prep_edges(srcF, dstF, EFp, Np)
  bF = jnp.concatenate([left_batch, right_batch + G], axis=0)
  bF = _pad_ids(bF, Np, 2 * G).reshape(Np // CH, CH)

  degL = _sc_hist(Np, Ep // CH)(dstL)
  degF = _sc_hist(Np, EFp // CH)(dstF)
  cntF = _sc_hist(PoolR, Np // CH)(bF)
  invLe, invLo = _tc_invdeg(Nh)(degL[0, 0::2], degL[1, 0::2],
                                degL[0, 1::2], degL[1, 1::2])
  invFe, invFo = _tc_invdeg(Nh)(degF[0, 0::2], degF[1, 0::2],
                                degF[0, 1::2], degF[1, 1::2])

  agg_f = _sc_agg(Np, EFp // CH)
  upd_f = _tc_update(Nh, with_ctx=False)
  hp = _tc_frag_in(Nh, IN)(xF, fp['in_W'], fp['in_b'])
  for i in range(len(fp['conv_W'])):
    agg = agg_f(hp.reshape(2 * Np, HALF), srcF, dstF)
    hp = upd_f(hp, agg.reshape(Nh, 4 * HALF), invFe, invFo,
               fp['conv_W'][i], fp['conv_b'][i],
               fp['ln_g'][i], fp['ln_b'][i])

  pooled = _sc_pool(Np, PoolR)(hp.reshape(Np, NC, HALF), bF)
  graph_ctx = _tc_pool_cond(PoolR, G, TD)(
      pooled[0:1], pooled[1:2], cntF[0], cntF[1], t,
      fp['out_W'], fp['out_b'],
      p['time_W1'], p['time_b1'], p['time_W2'], p['time_b2'],
      p['cond_W1'], p['cond_b1'], p['cond_W2'], p['cond_b2'])

  ctx = _sc_gather_rows(Np, 2 * HALF)(graph_ctx, batL)
  ctxp = ctx.reshape(Nh, 4 * HALF)
  hp = _tc_linker_in(Nh, IN)(xL, ntLe, ntLo, ctxp, p['in_W'], p['in_b'])
  agg_l = _sc_agg(Np, Ep // CH)
  upd_l = _tc_update(Nh, with_ctx=True)
  for i in range(len(p['conv_W'])):
    agg = agg_l(hp.reshape(2 * Np, HALF), srcL, dstL)
    hp = upd_l(hp, agg.reshape(Nh, 4 * HALF), invLe, invLo, ctxp,
               p['conv_W'][i], p['conv_b'][i],
               p['ln_g'][i], p['ln_b'][i])

  out = _tc_out(Nh, p['out_W'].shape[1])(hp, p['out_W'], p['out_b'])
  OUTD = p['out_W'].shape[1]
  return out.reshape(Np, OUTD)[:N][None]


# speedup vs baseline: 1.9223x; 1.0002x over previous
import functools
import math

import jax
import jax.numpy as jnp
from jax import lax
from jax.experimental import pallas as pl
from jax.experimental.pallas import tpu as pltpu
from jax.experimental.pallas import tpu_sc as plsc

NC = 2
NS = 16
LANES = 16
CH = 128
MACRO = 4
HALF = 32

_f32 = jnp.float32


def _ceil_to(x, m):
  return (x + m - 1) // m * m


def _zero_fill(buf_ref, nrows, ncols):
  per_row = ncols // LANES

  def body(i, _):
    r = i // per_row
    c = (i % per_row) * LANES
    buf_ref[r, pl.ds(c, LANES)] = jnp.zeros((LANES,), _f32)
    return 0

  lax.fori_loop(0, nrows * per_row, body, 0)


def _sc_hist(np_rows, n_chunks):
  mesh = plsc.VectorSubcoreMesh(core_axis_name="c", subcore_axis_name="s",
                                num_cores=NC, num_subcores=NS)
  acc_rows = np_rows + CH
  out_chunks = np_rows // CH

  @functools.partial(
      pl.kernel,
      mesh=mesh,
      compiler_params=pltpu.CompilerParams(use_tc_tiling_on_sc=False),
      out_type=jax.ShapeDtypeStruct((NC, np_rows), _f32),
      scratch_types=[
          pltpu.VMEM_SHARED((acc_rows,), _f32),
          pltpu.VMEM((CH,), jnp.int32),
          pltpu.VMEM((CH,), _f32),
          pltpu.VMEM((CH,), _f32),
      ],
  )
  def k(dst_hbm, out_hbm, acc, idx_v, ones_v, buf_v):
    c = lax.axis_index("c")
    s = lax.axis_index("s")

    def fill(i, _):
      buf_v[pl.ds(i * LANES, LANES)] = jnp.zeros((LANES,), _f32)
      ones_v[pl.ds(i * LANES, LANES)] = jnp.ones((LANES,), _f32)
      return 0

    lax.fori_loop(0, CH // LANES, fill, 0)

    nz = acc_rows // CH

    def zbody(i, _):
      pltpu.sync_copy(buf_v, acc.at[pl.ds((i * NS + s) * CH, CH)])
      return 0

    lax.fori_loop(0, nz // NS, zbody, 0)

    @pl.when(s < nz % NS)
    def _():
      pltpu.sync_copy(buf_v, acc.at[pl.ds(((nz // NS) * NS + s) * CH, CH)])

    plsc.subcore_barrier()

    half = n_chunks // NC

    def ebody(i, _):
      ch = c * half + i * NS + s
      pltpu.sync_copy(dst_hbm.at[ch], idx_v)
      pltpu.sync_copy(ones_v, acc.at[idx_v], add=True)
      return 0

    lax.fori_loop(0, half // NS, ebody, 0)

    @pl.when(s < half % NS)
    def _():
      ch = c * half + (half // NS) * NS + s
      pltpu.sync_copy(dst_hbm.at[ch], idx_v)
      pltpu.sync_copy(ones_v, acc.at[idx_v], add=True)

    plsc.subcore_barrier()

    def obody(i, _):
      r = (i * NS + s) * CH
      pltpu.sync_copy(acc.at[pl.ds(r, CH)], buf_v)
      pltpu.sync_copy(buf_v, out_hbm.at[c, pl.ds(r, CH)])
      return 0

    lax.fori_loop(0, out_chunks // NS, obody, 0)

    @pl.when(s < out_chunks % NS)
    def _():
      r = ((out_chunks // NS) * NS + s) * CH
      pltpu.sync_copy(acc.at[pl.ds(r, CH)], buf_v)
      pltpu.sync_copy(buf_v, out_hbm.at[c, pl.ds(r, CH)])

  return k


def _sc_agg(np_rows, n_chunks):
  mesh = plsc.VectorSubcoreMesh(core_axis_name="c", subcore_axis_name="s",
                                num_cores=NC, num_subcores=NS)
  acc_rows = np_rows + CH
  out_chunks = np_rows // CH

  @functools.partial(
      pl.kernel,
      mesh=mesh,
      compiler_params=pltpu.CompilerParams(use_tc_tiling_on_sc=False),
      out_type=jax.ShapeDtypeStruct((np_rows, NC, HALF), _f32),
      scratch_types=[
          pltpu.VMEM_SHARED((acc_rows, HALF), _f32),
          pltpu.VMEM((MACRO, CH), jnp.int32),
          pltpu.VMEM((MACRO, CH), jnp.int32),
          pltpu.VMEM((MACRO, CH), jnp.int32),
          pltpu.VMEM((MACRO, CH), jnp.int32),
          pltpu.VMEM((MACRO * CH, HALF), _f32),
          pltpu.SemaphoreType.DMA,
          pltpu.SemaphoreType.DMA,
      ],
  )
  def k(h_hbm, src_hbm, dst_hbm, out_hbm, acc, src_a, dst_a, src_b, dst_b,
        rows_v, gsem, ssem):
    c = lax.axis_index("c")
    s = lax.axis_index("s")
    buf_v = rows_v.at[pl.ds(0, CH)]
    _zero_fill(rows_v, CH, HALF)

    nz = acc_rows // CH

    def zbody(i, _):
      pltpu.sync_copy(buf_v, acc.at[pl.ds((i * NS + s) * CH, CH)])
      return 0

    lax.fori_loop(0, nz // NS, zbody, 0)

    @pl.when(s < nz % NS)
    def _():
      pltpu.sync_copy(buf_v, acc.at[pl.ds(((nz // NS) * NS + s) * CH, CH)])

    plsc.subcore_barrier()

    n_tile_macros = n_chunks // MACRO // NS
    max_m = (n_chunks // MACRO - 1) * MACRO

    def run_macro(src_i):
      ga = [pltpu.async_copy(h_hbm.at[src_i.at[j]],
                             rows_v.at[pl.ds(j * CH, CH)], gsem)
            for j in range(MACRO)]
      return ga

    def load_idx(k, src_i, dst_i):
      m = jnp.minimum((k * NS + s) * MACRO, max_m)
      pltpu.sync_copy(src_hbm.at[pl.ds(m, MACRO)], src_i)
      pltpu.sync_copy(dst_hbm.at[pl.ds(m, MACRO)], dst_i)
      for j in range(MACRO):
        for t in range(CH // LANES):
          v = src_i[j, pl.ds(t * LANES, LANES)]
          src_i[j, pl.ds(t * LANES, LANES)] = v * 2 + c

    def drain_scatter(dst_i):
      sa = [pltpu.async_copy(rows_v.at[pl.ds(j * CH, CH)],
                             acc.at[dst_i.at[j]], ssem, add=True)
            for j in range(MACRO)]
      for d in sa:
        d.wait()

    load_idx(0, src_a, dst_a)

    def ebody(i, _):
      ga = run_macro(src_a)
      load_idx(2 * i + 1, src_b, dst_b)
      for d in ga:
        d.wait()
      drain_scatter(dst_a)
      gb = run_macro(src_b)
      load_idx(2 * i + 2, src_a, dst_a)
      for d in gb:
        d.wait()
      drain_scatter(dst_b)
      return 0

    lax.fori_loop(0, n_tile_macros // 2, ebody, 0)

    plsc.subcore_barrier()

    def obody(i, _):
      r = (i * NS + s) * CH
      pltpu.sync_copy(acc.at[pl.ds(r, CH)], buf_v)
      pltpu.sync_copy(buf_v, out_hbm.at[pl.ds(r, CH), c])
      return 0

    lax.fori_loop(0, out_chunks // NS, obody, 0)

    @pl.when(s < out_chunks % NS)
    def _():
      r = ((out_chunks // NS) * NS + s) * CH
      pltpu.sync_copy(acc.at[pl.ds(r, CH)], buf_v)
      pltpu.sync_copy(buf_v, out_hbm.at[pl.ds(r, CH), c])

  return k


def _sc_pool(np_rows, pool_rows):
  mesh = plsc.VectorSubcoreMesh(core_axis_name="c", subcore_axis_name="s",
                                num_cores=NC, num_subcores=NS)
  n_chunks = np_rows // CH
  out_chunks = pool_rows // CH

  @functools.partial(
      pl.kernel,
      mesh=mesh,
      compiler_params=pltpu.CompilerParams(use_tc_tiling_on_sc=False),
      out_type=jax.ShapeDtypeStruct((NC, pool_rows, HALF), _f32),
      scratch_types=[
          pltpu.VMEM_SHARED((pool_rows, HALF), _f32),
          pltpu.VMEM((CH,), jnp.int32),
          pltpu.VMEM((CH, HALF), _f32),
          pltpu.VMEM((CH, HALF), _f32),
      ],
  )
  def k(h_hbm, b_hbm, out_hbm, acc, idx_v, vals_v, buf_v):
    c = lax.axis_index("c")
    s = lax.axis_index("s")
    _zero_fill(buf_v, CH, HALF)

    @pl.when(s < out_chunks)
    def _():
      pltpu.sync_copy(buf_v, acc.at[pl.ds(s * CH, CH)])

    plsc.subcore_barrier()

    def ebody(i, _):
      ch = i * NS + s
      pltpu.sync_copy(b_hbm.at[ch], idx_v)
      pltpu.sync_copy(h_hbm.at[pl.ds(ch * CH, CH), c], vals_v)
      pltpu.sync_copy(vals_v, acc.at[idx_v], add=True)
      return 0

    lax.fori_loop(0, n_chunks // NS, ebody, 0)

    @pl.when(s < n_chunks % NS)
    def _():
      ch = (n_chunks // NS) * NS + s
      pltpu.sync_copy(b_hbm.at[ch], idx_v)
      pltpu.sync_copy(h_hbm.at[pl.ds(ch * CH, CH), c], vals_v)
      pltpu.sync_copy(vals_v, acc.at[idx_v], add=True)

    plsc.subcore_barrier()

    @pl.when(s < out_chunks)
    def _():
      pltpu.sync_copy(acc.at[pl.ds(s * CH, CH)], buf_v)
      pltpu.sync_copy(buf_v, out_hbm.at[c, pl.ds(s * CH, CH)])

  return k


def _sc_gather_rows(np_rows, d):
  mesh = plsc.VectorSubcoreMesh(core_axis_name="c", subcore_axis_name="s",
                                num_cores=NC, num_subcores=NS)
  n_chunks = np_rows // CH

  @functools.partial(
      pl.kernel,
      mesh=mesh,
      compiler_params=pltpu.CompilerParams(use_tc_tiling_on_sc=False),
      out_type=jax.ShapeDtypeStruct((np_rows, d), _f32),
      scratch_types=[
          pltpu.VMEM((CH,), jnp.int32),
          pltpu.VMEM((CH, d), _f32),
          pltpu.SemaphoreType.DMA,
      ],
  )
  def k(tab_hbm, idx_hbm, out_hbm, idx_v, rows_v, sem):
    c = lax.axis_index("c")
    s = lax.axis_index("s")
    w = s * NC + c
    nw = NC * NS

    def body(i, _):
      ch = i * nw + w
      pltpu.sync_copy(idx_hbm.at[ch], idx_v)
      pltpu.async_copy(tab_hbm.at[idx_v], rows_v, sem).wait()
      pltpu.sync_copy(rows_v, out_hbm.at[pl.ds(ch * CH, CH)])
      return 0

    lax.fori_loop(0, n_chunks // nw, body, 0)

  return k


_BLK = 1024


def _tc_invdeg(nh_rows):
  blk = 1024

  def body(d0e_ref, d1e_ref, d0o_ref, d1o_ref, oe_ref, oo_ref):
    oe_ref[...] = 1.0 / jnp.maximum(d0e_ref[...] + d1e_ref[...], 1.0)
    oo_ref[...] = 1.0 / jnp.maximum(d0o_ref[...] + d1o_ref[...], 1.0)

  sp = pl.BlockSpec((blk,), lambda i: (i,))
  return pl.pallas_call(
      body,
      grid=(nh_rows // blk,),
      in_specs=[sp, sp, sp, sp],
      out_specs=[sp, sp],
      out_shape=[jax.ShapeDtypeStruct((nh_rows,), _f32),
                 jax.ShapeDtypeStruct((nh_rows,), _f32)],
  )


def _tc_frag_in(nh_rows, in_dim):
  h = 2 * HALF

  def body(x_ref, w_ref, b_ref, o_ref):
    w = w_ref[...]
    bb = b_ref[...][None, :]
    x = x_ref[...]
    o_ref[:, :h] = jnp.dot(x[:, :in_dim], w,
                           preferred_element_type=_f32) + bb
    o_ref[:, h:] = jnp.dot(x[:, in_dim:], w,
                           preferred_element_type=_f32) + bb

  return pl.pallas_call(
      body,
      grid=(nh_rows // _BLK,),
      in_specs=[
          pl.BlockSpec((_BLK, 2 * in_dim), lambda i: (i, 0)),
          pl.BlockSpec((in_dim, h), lambda i: (0, 0)),
          pl.BlockSpec((h,), lambda i: (0,)),
      ],
      out_specs=pl.BlockSpec((_BLK, 2 * h), lambda i: (i, 0)),
      out_shape=jax.ShapeDtypeStruct((nh_rows, 2 * h), _f32),
  )


def _layer_norm_rows(y, g, b):
  m = jnp.mean(y, axis=-1, keepdims=True)
  v = jnp.mean((y - m) * (y - m), axis=-1, keepdims=True)
  return (y - m) / jnp.sqrt(v + 1e-5) * g[None, :] + b[None, :]


def _tc_update(nh_rows, with_ctx):
  h = 2 * HALF

  def body(*refs):
    if with_ctx:
      (h_ref, a_ref, ie_ref, io_ref, ctx_ref, w_ref, b_ref,
       g_ref, bl_ref, o_ref) = refs
    else:
      (h_ref, a_ref, ie_ref, io_ref, w_ref, b_ref, g_ref, bl_ref,
       o_ref) = refs
    w = w_ref[...]
    bb = b_ref[...][None, :]
    hp = h_ref[...]
    ap = a_ref[...]

    def half_update(xh, ah, inv, ctxh):
      x = xh + ah * inv[:, None]
      y = (jnp.dot(x[:, :HALF], w[:HALF, :], preferred_element_type=_f32)
           + jnp.dot(x[:, HALF:], w[HALF:, :], preferred_element_type=_f32)
           + bb)
      if with_ctx:
        y = y + ctxh
      y = _layer_norm_rows(y, g_ref[...], bl_ref[...])
      return jnp.maximum(y, 0.0)

    ctx_lo = ctx_ref[...][:, :h] if with_ctx else None
    ctx_hi = ctx_ref[...][:, h:] if with_ctx else None
    o_ref[:, :h] = half_update(hp[:, :h], ap[:, :h], ie_ref[...], ctx_lo)
    o_ref[:, h:] = half_update(hp[:, h:], ap[:, h:], io_ref[...], ctx_hi)

  in_specs = [
      pl.BlockSpec((_BLK, 2 * h), lambda i: (i, 0)),
      pl.BlockSpec((_BLK, 2 * h), lambda i: (i, 0)),
      pl.BlockSpec((_BLK,), lambda i: (i,)),
      pl.BlockSpec((_BLK,), lambda i: (i,)),
  ]
  if with_ctx:
    in_specs.append(pl.BlockSpec((_BLK, 2 * h), lambda i: (i, 0)))
  in_specs += [
      pl.BlockSpec((h, h), lambda i: (0, 0)),
      pl.BlockSpec((h,), lambda i: (0,)),
      pl.BlockSpec((h,), lambda i: (0,)),
      pl.BlockSpec((h,), lambda i: (0,)),
  ]
  return pl.pallas_call(
      body,
      grid=(nh_rows // _BLK,),
      in_specs=in_specs,
      out_specs=pl.BlockSpec((_BLK, 2 * h), lambda i: (i, 0)),
      out_shape=jax.ShapeDtypeStruct((nh_rows, 2 * h), _f32),
  )


def _tc_pool_cond(pool_rows, g_count, td):
  h = 2 * HALF

  def body(p0_ref, p1_ref, c0_ref, c1_ref, t_ref, fw_ref, fb_ref,
           tw1_ref, tb1_ref, tw2_ref, tb2_ref, cw1_ref, cb1_ref,
           cw2_ref, cb2_ref, o_ref):
    pool = jnp.concatenate([p0_ref[0], p1_ref[0]], axis=1)
    cnt = (c0_ref[...] + c1_ref[...])[:, None]
    mean = pool / jnp.maximum(cnt, 1.0)
    fo = jnp.dot(mean, fw_ref[...],
                 preferred_element_type=_f32) + fb_ref[...][None, :]
    left = fo[:g_count, :]
    right = fo[g_count:2 * g_count, :]
    half = td // 2
    i = lax.broadcasted_iota(jnp.int32, (g_count, half), 1).astype(_f32)
    freqs = jnp.exp((-math.log(10000.0) / half) * i)
    a = t_ref[...][:, None] * freqs
    te = jnp.concatenate([jnp.sin(a), jnp.cos(a)], axis=1)
    th = jnp.dot(te, tw1_ref[...],
                 preferred_element_type=_f32) + tb1_ref[...][None, :]
    th = th * jax.nn.sigmoid(th)
    th = jnp.dot(th, tw2_ref[...],
                 preferred_element_type=_f32) + tb2_ref[...][None, :]
    ci = jnp.concatenate([left, right, th], axis=1)
    gc = jnp.dot(ci, cw1_ref[...],
                 preferred_element_type=_f32) + cb1_ref[...][None, :]
    gc = gc * jax.nn.sigmoid(gc)
    gc = jnp.dot(gc, cw2_ref[...],
                 preferred_element_type=_f32) + cb2_ref[...][None, :]
    o_ref[...] = gc

  full = lambda *shape: pl.BlockSpec(shape, lambda: tuple(0 for _ in shape))
  return pl.pallas_call(
      body,
      in_specs=[
          full(1, pool_rows, HALF), full(1, pool_rows, HALF),
          full(pool_rows), full(pool_rows),
          full(g_count),
          full(h, h), full(h),
          full(td, h), full(h), full(h, h), full(h),
          full(3 * h, h), full(h), full(h, h), full(h),
      ],
      out_specs=full(g_count, h),
      out_shape=jax.ShapeDtypeStruct((g_count, h), _f32),
  )


def _tc_linker_in(nh_rows, in_dim):
  h = 2 * HALF

  def body(x_ref, nte_ref, nto_ref, ctx_ref, w_ref, b_ref, o_ref):
    w = w_ref[...]
    bb = b_ref[...][None, :]
    x = x_ref[...]
    ctx = ctx_ref[...]

    def half_in(xh, nt, ctxh):
      y = jnp.dot(xh, w[:in_dim, :], preferred_element_type=_f32) + bb
      ntc = jnp.clip(nt, 0, 2)
      for kcls in range(3):
        y = y + (ntc == kcls).astype(_f32)[:, None] * w[in_dim + kcls][None]
      y = y + (nt > 0).astype(_f32)[:, None] * w[in_dim + 3][None]
      return y + ctxh

    o_ref[:, :h] = half_in(x[:, :in_dim], nte_ref[...], ctx[:, :h])
    o_ref[:, h:] = half_in(x[:, in_dim:], nto_ref[...], ctx[:, h:])

  return pl.pallas_call(
      body,
      grid=(nh_rows // _BLK,),
      in_specs=[
          pl.BlockSpec((_BLK, 2 * in_dim), lambda i: (i, 0)),
          pl.BlockSpec((_BLK,), lambda i: (i,)),
          pl.BlockSpec((_BLK,), lambda i: (i,)),
          pl.BlockSpec((_BLK, 2 * h), lambda i: (i, 0)),
          pl.BlockSpec((in_dim + 4, h), lambda i: (0, 0)),
          pl.BlockSpec((h,), lambda i: (0,)),
      ],
      out_specs=pl.BlockSpec((_BLK, 2 * h), lambda i: (i, 0)),
      out_shape=jax.ShapeDtypeStruct((nh_rows, 2 * h), _f32),
  )


def _tc_out(nh_rows, out_dim):
  h = 2 * HALF

  def body(h_ref, w_ref, b_ref, o_ref):
    w = w_ref[...]
    bb = b_ref[...][None, :]
    hp = h_ref[...]

    def half_out(xh):
      return (jnp.dot(xh[:, :HALF], w[:HALF, :],
                      preferred_element_type=_f32)
              + jnp.dot(xh[:, HALF:], w[HALF:, :],
                        preferred_element_type=_f32) + bb)

    o_ref[:, :out_dim] = half_out(hp[:, :h])
    o_ref[:, out_dim:] = half_out(hp[:, h:])

  return pl.pallas_call(
      body,
      grid=(nh_rows // _BLK,),
      in_specs=[
          pl.BlockSpec((_BLK, 2 * h), lambda i: (i, 0)),
          pl.BlockSpec((h, out_dim), lambda i: (0, 0)),
          pl.BlockSpec((out_dim,), lambda i: (0,)),
      ],
      out_specs=pl.BlockSpec((_BLK, 2 * out_dim), lambda i: (i, 0)),
      out_shape=jax.ShapeDtypeStruct((nh_rows, 2 * out_dim), _f32),
  )


def _pad_nodes_2d(a, np_rows, fill=0.0):
  return jnp.pad(a, ((0, np_rows - a.shape[0]), (0, 0)),
                 constant_values=fill)


def _pad_ids(ids, np_rows, fill):
  return jnp.pad(ids.astype(jnp.int32), (0, np_rows - ids.shape[0]),
                 constant_values=fill)


def _prep_edges(src, dst, ep, dummy_dst):
  e = src.shape[0]
  src = jnp.pad(src.astype(jnp.int32), (0, ep - e), constant_values=0)
  dst = jnp.pad(dst.astype(jnp.int32), (0, ep - e),
                constant_values=dummy_dst)
  return src.reshape(ep // CH, CH), dst.reshape(ep // CH, CH)


def kernel(x, t, linker_batch, linker_graph_ptr, linker_node_type,
           linker_edge_index, left_x, left_edge_index, left_batch,
           right_x, right_edge_index, right_batch, params):
  G = int(linker_graph_ptr.shape[0]) - 1
  N = x.shape[1]
  IN = x.shape[2]
  NF = left_x.shape[0]
  E = linker_edge_index.shape[1]
  EF = left_edge_index.shape[1]
  TD = params['time_W1'].shape[0]

  Np = _ceil_to(max(N, 2 * NF), NC * NS * CH)
  e_unit = NS * CH * 2 * MACRO
  Ep = _ceil_to(E, e_unit)
  EFp = _ceil_to(2 * EF, e_unit)
  PoolR = _ceil_to(2 * G + 1, CH)

  p = params
  fp = p['frag']
  Nh = Np // 2

  xL = _pad_nodes_2d(x[0], Np).reshape(Nh, 2 * IN)
  ntL = _pad_ids(linker_node_type, Np, 0)
  ntLe, ntLo = ntL[0::2], ntL[1::2]
  batL = _pad_ids(linker_batch, Np, 0).reshape(Np // CH, CH)
  srcL, dstL = _prep_edges(linker_edge_index[0], linker_edge_index[1],
                           Ep, Np)

  xF = _pad_nodes_2d(jnp.concatenate([left_x, right_x], axis=0),
                     Np).reshape(Nh, 2 * IN)
  srcF = jnp.concatenate([left_edge_index[0],
                          right_edge_index[0] + NF], axis=0)
  dstF = jnp.concatenate([left_edge_index[1],
                          right_edge_index[1] + NF], axis=0)
  srcF, dstF = _prep_edges(srcF, dstF, EFp, Np)
  bF = jnp.concatenate([left_batch, right_batch + G], axis=0)
  bF = _pad_ids(bF, Np, 2 * G).reshape(Np // CH, CH)

  degL = _sc_hist(Np, Ep // CH)(dstL)
  degF = _sc_hist(Np, EFp // CH)(dstF)
  cntF = _sc_hist(PoolR, Np // CH)(bF)
  invLe, invLo = _tc_invdeg(Nh)(degL[0, 0::2], degL[1, 0::2],
                                degL[0, 1::2], degL[1, 1::2])
  invFe, invFo = _tc_invdeg(Nh)(degF[0, 0::2], degF[1, 0::2],
                                degF[0, 1::2], degF[1, 1::2])

  agg_f = _sc_agg(Np, EFp // CH)
  upd_f = _tc_update(Nh, with_ctx=False)
  hp = _tc_frag_in(Nh, IN)(xF, fp['in_W'], fp['in_b'])
  for i in range(len(fp['conv_W'])):
    agg = agg_f(hp.reshape(2 * Np, HALF), srcF, dstF)
    hp = upd_f(hp, agg.reshape(Nh, 4 * HALF), invFe, invFo,
               fp['conv_W'][i], fp['conv_b'][i],
               fp['ln_g'][i], fp['ln_b'][i])

  pooled = _sc_pool(Np, PoolR)(hp.reshape(Np, NC, HALF), bF)
  graph_ctx = _tc_pool_cond(PoolR, G, TD)(
      pooled[0:1], pooled[1:2], cntF[0], cntF[1], t,
      fp['out_W'], fp['out_b'],
      p['time_W1'], p['time_b1'], p['time_W2'], p['time_b2'],
      p['cond_W1'], p['cond_b1'], p['cond_W2'], p['cond_b2'])

  ctx = _sc_gather_rows(Np, 2 * HALF)(graph_ctx, batL)
  ctxp = ctx.reshape(Nh, 4 * HALF)
  hp = _tc_linker_in(Nh, IN)(xL, ntLe, ntLo, ctxp, p['in_W'], p['in_b'])
  agg_l = _sc_agg(Np, Ep // CH)
  upd_l = _tc_update(Nh, with_ctx=True)
  for i in range(len(p['conv_W'])):
    agg = agg_l(hp.reshape(2 * Np, HALF), srcL, dstL)
    hp = upd_l(hp, agg.reshape(Nh, 4 * HALF), invLe, invLo, ctxp,
               p['conv_W'][i], p['conv_b'][i],
               p['ln_g'][i], p['ln_b'][i])

  out = _tc_out(Nh, p['out_W'].shape[1])(hp, p['out_W'], p['out_b'])
  OUTD = p['out_W'].shape[1]
  return out.reshape(Np, OUTD)[:N][None]
